# Initial kernel scaffold; baseline (speedup 1.0000x reference)
import jax
import jax.numpy as jnp
from jax.experimental import pallas as pl


def kernel(vertices, onehot, dir0, w1, b1, dir1, w2, b2, dir2, w3, b3, dir3, w4, b4, dir4, w5, b5, dir5, w6, b6, dir6, w7, b7, dir7, W1, B1, W2, B2, W3, B3):
    raise NotImplementedError("write your pallas kernel here")


# speedup vs baseline: 1.4467x; 1.4467x over previous
import functools

import jax
import jax.numpy as jnp
from jax import lax
from jax.experimental import pallas as pl

SUP = 7
OUTC = 32
NBR = 16

_INTERPRET = False


def _pc(body, grid, in_specs, out_specs, out_shape):
    return pl.pallas_call(
        body,
        grid=grid,
        in_specs=in_specs,
        out_specs=out_specs,
        out_shape=out_shape,
        interpret=_INTERPRET,
    )


def _ball_body(q_ref, xT_ref, ni_ref, *, r2, nsample, V):
    q = q_ref[0]
    xT = xT_ref[0]
    d = ((q[:, 0:1] - xT[0:1, :]) ** 2
         + (q[:, 1:2] - xT[1:2, :]) ** 2
         + (q[:, 2:3] - xT[2:3, :]) ** 2)
    Qb = q.shape[0]
    iota = lax.broadcasted_iota(jnp.int32, (Qb, V), 1)
    val = jnp.where(d > r2, V, iota)
    cols = jnp.full((Qb, nsample), V, jnp.int32)
    kiota = lax.broadcasted_iota(jnp.int32, (Qb, nsample), 1)
    for k in range(nsample):
        m = jnp.min(val, axis=1, keepdims=True)
        cols = jnp.where(kiota == k, m, cols)
        val = jnp.where(val == m, V, val)
    first = cols[:, 0:1]
    ni_ref[0] = jnp.where(cols == V, first, cols)


def _ball(queries, xT, radius, nsample, qb):
    B, S, _ = queries.shape
    V = xT.shape[2]
    body = functools.partial(_ball_body, r2=radius * radius, nsample=nsample, V=V)
    return _pc(
        body,
        grid=(B, S // qb),
        in_specs=[
            pl.BlockSpec((1, qb, 3), lambda b, i: (b, i, 0)),
            pl.BlockSpec((1, 3, V), lambda b, i: (b, 0, 0)),
        ],
        out_specs=pl.BlockSpec((1, qb, nsample), lambda b, i: (b, i, 0)),
        out_shape=jax.ShapeDtypeStruct((B, S, nsample), jnp.int32),
    )(queries, xT)


def _conv_surface_body(ni_ref, q_ref, xyz_ref, sup_ref, out_ref, *, V):
    q = q_ref[0]
    xyz = xyz_ref[0]
    sup = sup_ref[...]
    supn = sup / jnp.sqrt(jnp.sum(sup * sup, axis=0, keepdims=True) + 1e-12)
    Qb = q.shape[0]
    iota = lax.broadcasted_iota(jnp.int32, (Qb, V), 1)
    ni = ni_ref[0]
    acc = None
    for k in range(NBR):
        sel = (iota == ni[:, k:k + 1]).astype(jnp.float32)
        nbr = jnp.dot(sel, xyz, preferred_element_type=jnp.float32, precision=lax.Precision.HIGHEST)
        dirv = nbr - q
        dn = dirv / jnp.sqrt(jnp.sum(dirv * dirv, axis=1, keepdims=True) + 1e-12)
        theta = jnp.maximum(jnp.dot(dn, supn, preferred_element_type=jnp.float32, precision=lax.Precision.HIGHEST), 0.0)
        acc = theta if acc is None else jnp.maximum(acc, theta)
    s = acc[:, 0:OUTC]
    for si in range(1, SUP):
        s = s + acc[:, si * OUTC:(si + 1) * OUTC]
    out_ref[0] = jnp.maximum(s, 0.0)


def _conv_surface(ni, xyz, sup, qb):
    B, S, _ = xyz.shape
    V = S
    body = functools.partial(_conv_surface_body, V=V)
    return _pc(
        body,
        grid=(B, S // qb),
        in_specs=[
            pl.BlockSpec((1, qb, NBR), lambda b, i: (b, i, 0)),
            pl.BlockSpec((1, qb, 3), lambda b, i: (b, i, 0)),
            pl.BlockSpec((1, V, 3), lambda b, i: (b, 0, 0)),
            pl.BlockSpec((3, SUP * OUTC), lambda b, i: (0, 0)),
        ],
        out_specs=pl.BlockSpec((1, qb, OUTC), lambda b, i: (b, i, 0)),
        out_shape=jax.ShapeDtypeStruct((B, S, OUTC), jnp.float32),
    )(ni, xyz, xyz, sup)


def _conv_body(ni_ref, q_ref, xyz_ref, fo_ref, foblk_ref, sup_ref, out_ref, *, V):
    q = q_ref[0]
    xyz = xyz_ref[0]
    sup = sup_ref[...]
    supn = sup / jnp.sqrt(jnp.sum(sup * sup, axis=0, keepdims=True) + 1e-12)
    Qb = q.shape[0]
    iota = lax.broadcasted_iota(jnp.int32, (Qb, V), 1)
    ni = ni_ref[0]
    ftail = fo_ref[0][:, OUTC:]
    acc = None
    for k in range(NBR):
        sel = (iota == ni[:, k:k + 1]).astype(jnp.float32)
        nbr = jnp.dot(sel, xyz, preferred_element_type=jnp.float32, precision=lax.Precision.HIGHEST)
        dirv = nbr - q
        dn = dirv / jnp.sqrt(jnp.sum(dirv * dirv, axis=1, keepdims=True) + 1e-12)
        theta = jnp.maximum(jnp.dot(dn, supn, preferred_element_type=jnp.float32, precision=lax.Precision.HIGHEST), 0.0)
        fs = jnp.dot(sel, ftail, preferred_element_type=jnp.float32, precision=lax.Precision.HIGHEST)
        a = theta * fs
        acc = a if acc is None else jnp.maximum(acc, a)
    s = acc[:, 0:OUTC]
    for si in range(1, SUP):
        s = s + acc[:, si * OUTC:(si + 1) * OUTC]
    out_ref[0] = jnp.maximum(s + foblk_ref[0][:, 0:OUTC], 0.0)


def _conv(ni, xyz, fo, sup, qb):
    B, S, _ = xyz.shape
    V = S
    C = fo.shape[2]
    body = functools.partial(_conv_body, V=V)
    return _pc(
        body,
        grid=(B, S // qb),
        in_specs=[
            pl.BlockSpec((1, qb, NBR), lambda b, i: (b, i, 0)),
            pl.BlockSpec((1, qb, 3), lambda b, i: (b, i, 0)),
            pl.BlockSpec((1, V, 3), lambda b, i: (b, 0, 0)),
            pl.BlockSpec((1, V, C), lambda b, i: (b, 0, 0)),
            pl.BlockSpec((1, qb, C), lambda b, i: (b, i, 0)),
            pl.BlockSpec((3, SUP * OUTC), lambda b, i: (0, 0)),
        ],
        out_specs=pl.BlockSpec((1, qb, OUTC), lambda b, i: (b, i, 0)),
        out_shape=jax.ShapeDtypeStruct((B, S, OUTC), jnp.float32),
    )(ni, xyz, xyz, fo, fo, sup)


def _matmul_body(x_ref, w_ref, b_ref, o_ref):
    o_ref[0] = (jnp.dot(x_ref[0], w_ref[...], preferred_element_type=jnp.float32, precision=lax.Precision.HIGHEST)
                + b_ref[...])


def _matmul(x, w, b):
    B, S, Cin = x.shape
    Cout = w.shape[1]
    return _pc(
        _matmul_body,
        grid=(B,),
        in_specs=[
            pl.BlockSpec((1, S, Cin), lambda b: (b, 0, 0)),
            pl.BlockSpec((Cin, Cout), lambda b: (0, 0)),
            pl.BlockSpec((1, Cout), lambda b: (0, 0)),
        ],
        out_specs=pl.BlockSpec((1, S, Cout), lambda b: (b, 0, 0)),
        out_shape=jax.ShapeDtypeStruct((B, S, Cout), jnp.float32),
    )(x, w, b.reshape(1, Cout))


def _poolmax_body(ni_ref, feat_ref, o_ref, *, V, nsample):
    ni = ni_ref[0]
    feat = feat_ref[0]
    Sb = ni.shape[0]
    iota = lax.broadcasted_iota(jnp.int32, (Sb, V), 1)
    out = None
    for k in range(nsample):
        sel = (iota == ni[:, k:k + 1]).astype(jnp.float32)
        g = jnp.dot(sel, feat, preferred_element_type=jnp.float32, precision=lax.Precision.HIGHEST)
        out = g if out is None else jnp.maximum(out, g)
    o_ref[0] = out


def _poolmax(ni, feat, qb):
    B, S, nsample = ni.shape
    V, C = feat.shape[1], feat.shape[2]
    body = functools.partial(_poolmax_body, V=V, nsample=nsample)
    return _pc(
        body,
        grid=(B, S // qb),
        in_specs=[
            pl.BlockSpec((1, qb, nsample), lambda b, i: (b, i, 0)),
            pl.BlockSpec((1, V, C), lambda b, i: (b, 0, 0)),
        ],
        out_specs=pl.BlockSpec((1, qb, C), lambda b, i: (b, i, 0)),
        out_shape=jax.ShapeDtypeStruct((B, S, C), jnp.float32),
    )(ni, feat)


def _nearest_body(t_ref, sT_ref, o_ref, *, S):
    t = t_ref[0]
    sT = sT_ref[0]
    d = ((t[:, 0:1] - sT[0:1, :]) ** 2
         + (t[:, 1:2] - sT[1:2, :]) ** 2
         + (t[:, 2:3] - sT[2:3, :]) ** 2)
    Tb = t.shape[0]
    m = jnp.min(d, axis=1, keepdims=True)
    iota = lax.broadcasted_iota(jnp.int32, (Tb, S), 1)
    o_ref[0] = jnp.min(jnp.where(d == m, iota, S), axis=1, keepdims=True)


def _nearest(targets, sT, qb):
    B, T, _ = targets.shape
    S = sT.shape[2]
    body = functools.partial(_nearest_body, S=S)
    return _pc(
        body,
        grid=(B, T // qb),
        in_specs=[
            pl.BlockSpec((1, qb, 3), lambda b, i: (b, i, 0)),
            pl.BlockSpec((1, 3, S), lambda b, i: (b, 0, 0)),
        ],
        out_specs=pl.BlockSpec((1, qb, 1), lambda b, i: (b, i, 0)),
        out_shape=jax.ShapeDtypeStruct((B, T, 1), jnp.int32),
    )(targets, sT)


def _gather_body(idx_ref, feat_ref, o_ref, *, S):
    idx = idx_ref[0]
    feat = feat_ref[0]
    Tb = idx.shape[0]
    iota = lax.broadcasted_iota(jnp.int32, (Tb, S), 1)
    sel = (iota == idx).astype(jnp.float32)
    o_ref[0] = jnp.dot(sel, feat, preferred_element_type=jnp.float32, precision=lax.Precision.HIGHEST)


def _gather_rows(idx, feat, qb):
    B, T, _ = idx.shape
    S, C = feat.shape[1], feat.shape[2]
    body = functools.partial(_gather_body, S=S)
    return _pc(
        body,
        grid=(B, T // qb),
        in_specs=[
            pl.BlockSpec((1, qb, 1), lambda b, i: (b, i, 0)),
            pl.BlockSpec((1, S, C), lambda b, i: (b, 0, 0)),
        ],
        out_specs=pl.BlockSpec((1, qb, C), lambda b, i: (b, i, 0)),
        out_shape=jax.ShapeDtypeStruct((B, T, C), jnp.float32),
    )(idx, feat)


def _rowmax_body(x_ref, o_ref):
    o_ref[0] = jnp.max(x_ref[0], axis=0, keepdims=True)


def _rowmax(x):
    B, S, C = x.shape
    return _pc(
        _rowmax_body,
        grid=(B,),
        in_specs=[pl.BlockSpec((1, S, C), lambda b: (b, 0, 0))],
        out_specs=pl.BlockSpec((1, 1, C), lambda b: (b, 0, 0)),
        out_shape=jax.ShapeDtypeStruct((B, 1, C), jnp.float32),
    )(x)


def _mlp_body(x_ref, w1_ref, b1_ref, w2_ref, b2_ref, w3_ref, b3_ref, o_ref):
    h = jnp.maximum(jnp.dot(x_ref[0], w1_ref[...],
                            preferred_element_type=jnp.float32, precision=lax.Precision.HIGHEST) + b1_ref[...], 0.0)
    h = jnp.maximum(jnp.dot(h, w2_ref[...],
                            preferred_element_type=jnp.float32, precision=lax.Precision.HIGHEST) + b2_ref[...], 0.0)
    o_ref[0] = jnp.dot(h, w3_ref[...],
                       preferred_element_type=jnp.float32, precision=lax.Precision.HIGHEST) + b3_ref[...]


def _mlp(x, W1, B1, W2, B2, W3, B3, qb):
    B, T, C = x.shape
    H1, H2, CO = W1.shape[1], W2.shape[1], W3.shape[1]
    return _pc(
        _mlp_body,
        grid=(B, T // qb),
        in_specs=[
            pl.BlockSpec((1, qb, C), lambda b, i: (b, i, 0)),
            pl.BlockSpec((C, H1), lambda b, i: (0, 0)),
            pl.BlockSpec((1, H1), lambda b, i: (0, 0)),
            pl.BlockSpec((H1, H2), lambda b, i: (0, 0)),
            pl.BlockSpec((1, H2), lambda b, i: (0, 0)),
            pl.BlockSpec((H2, CO), lambda b, i: (0, 0)),
            pl.BlockSpec((1, CO), lambda b, i: (0, 0)),
        ],
        out_specs=pl.BlockSpec((1, qb, CO), lambda b, i: (b, i, 0)),
        out_shape=jax.ShapeDtypeStruct((B, T, CO), jnp.float32),
    )(x, W1, B1.reshape(1, H1), W2, B2.reshape(1, H2), W3, B3.reshape(1, CO))


def kernel(vertices, onehot, dir0, w1, b1, dir1, w2, b2, dir2, w3, b3, dir3,
           w4, b4, dir4, w5, b5, dir5, w6, b6, dir6, w7, b7, dir7,
           W1, B1, W2, B2, W3, B3):
    B, N, _ = vertices.shape
    xyz0 = vertices
    xT0 = jnp.transpose(xyz0, (0, 2, 1))

    ni0 = _ball(xyz0, xT0, 0.25, NBR, qb=256)
    fm0 = _conv_surface(ni0, xyz0, dir0, qb=256)
    fm1 = _conv(ni0, xyz0, _matmul(fm0, w1, b1), dir1, qb=256)
    fm1c = jnp.concatenate([fm0, fm1], axis=2)
    fm2 = _conv(ni0, xyz0, _matmul(fm1c, w2, b2), dir2, qb=256)
    fm2c = jnp.concatenate([fm1c, fm2], axis=2)

    vp1 = xyz0[:, ::4, :]
    fp1 = _poolmax(_ball(vp1, xT0, 0.25, 4, qb=512), fm2c, qb=512)
    xT1 = jnp.transpose(vp1, (0, 2, 1))
    ni1 = _ball(vp1, xT1, 0.39, NBR, qb=512)
    fm3 = _conv(ni1, vp1, _matmul(fp1, w3, b3), dir3, qb=512)
    fm3c = jnp.concatenate([fp1, fm3], axis=2)
    fm4 = _conv(ni1, vp1, _matmul(fm3c, w4, b4), dir4, qb=512)
    fm4c = jnp.concatenate([fm3c, fm4], axis=2)
    fm5 = _conv(ni1, vp1, _matmul(fm4c, w5, b5), dir5, qb=512)
    fm5c = jnp.concatenate([fm4c, fm5], axis=2)

    vp2 = vp1[:, ::4, :]
    fp2 = _poolmax(_ball(vp2, xT1, 0.39, 4, qb=128), fm5c, qb=128)
    xT2 = jnp.transpose(vp2, (0, 2, 1))
    ni2 = _ball(vp2, xT2, 0.63, NBR, qb=128)
    fm6 = _conv(ni2, vp2, _matmul(fp2, w6, b6), dir6, qb=128)
    fm6c = jnp.concatenate([fp2, fm6], axis=2)
    fm7 = _conv(ni2, vp2, _matmul(fm6c, w7, b7), dir7, qb=128)
    fm7c = jnp.concatenate([fm6c, fm7], axis=2)

    fglob = _rowmax(fm7c)
    np1 = _nearest(xyz0, xT1, qb=512)
    np2 = _nearest(xyz0, xT2, qb=512)
    upA = _gather_rows(np1, jnp.concatenate([fm3c, fm4c, fm5c], axis=2), qb=512)
    upB = _gather_rows(np2, jnp.concatenate([fm6c, fm7c], axis=2), qb=512)

    fuse = jnp.concatenate([
        fm0, fm1c, fm2c, upA, upB,
        jnp.broadcast_to(fglob, (B, N, fglob.shape[2])),
        jnp.broadcast_to(onehot[:, None, :], (B, N, onehot.shape[1])),
    ], axis=2)
    return _mlp(fuse, W1, B1, W2, B2, W3, B3, qb=512)


# speedup vs baseline: 6.3740x; 4.4058x over previous
import functools

import jax
import jax.numpy as jnp
from jax import lax
from jax.experimental import pallas as pl

SUP = 7
OUTC = 32
NBR = 16

_INTERPRET = False


def _split_bf16(t):
    t0 = t.astype(jnp.bfloat16)
    t1 = (t - t0.astype(jnp.float32)).astype(jnp.bfloat16)
    return t0, t1


def _gdot(sel, t0, t1):
    selb = sel.astype(jnp.bfloat16)
    return (jnp.dot(selb, t0, preferred_element_type=jnp.float32)
            + jnp.dot(selb, t1, preferred_element_type=jnp.float32))


def _split_bf16_3(t):
    t0 = t.astype(jnp.bfloat16)
    r1 = t - t0.astype(jnp.float32)
    t1 = r1.astype(jnp.bfloat16)
    t2 = (r1 - t1.astype(jnp.float32)).astype(jnp.bfloat16)
    return t0, t1, t2


def _gdot3(sel, t0, t1, t2):
    selb = sel.astype(jnp.bfloat16)
    return (jnp.dot(selb, t0, preferred_element_type=jnp.float32)
            + jnp.dot(selb, t1, preferred_element_type=jnp.float32)
            + jnp.dot(selb, t2, preferred_element_type=jnp.float32))


def _pc(body, grid, in_specs, out_specs, out_shape):
    return pl.pallas_call(
        body,
        grid=grid,
        in_specs=in_specs,
        out_specs=out_specs,
        out_shape=out_shape,
        interpret=_INTERPRET,
    )


def _ball_dn_body(q_ref, xT_ref, xyz_ref, ni_ref, dx_ref, dy_ref, dz_ref,
                  *, r2, nsample, V):
    q = q_ref[0]
    xT = xT_ref[0]
    d = ((q[:, 0:1] - xT[0:1, :]) ** 2
         + (q[:, 1:2] - xT[1:2, :]) ** 2
         + (q[:, 2:3] - xT[2:3, :]) ** 2)
    Qb = q.shape[0]
    iota = lax.broadcasted_iota(jnp.int32, (Qb, V), 1)
    val = jnp.where(d > r2, V, iota)
    cols = jnp.full((Qb, nsample), V, jnp.int32)
    kiota = lax.broadcasted_iota(jnp.int32, (Qb, nsample), 1)
    for k in range(nsample):
        m = jnp.min(val, axis=1, keepdims=True)
        cols = jnp.where(kiota == k, m, cols)
        val = jnp.where(val == m, V, val)
    first = cols[:, 0:1]
    cols = jnp.where(cols == V, first, cols)
    ni_ref[0] = cols

    xyz0c, xyz1c, xyz2c = _split_bf16_3(xyz_ref[0])
    dx = jnp.zeros((Qb, nsample), jnp.float32)
    dy = jnp.zeros((Qb, nsample), jnp.float32)
    dz = jnp.zeros((Qb, nsample), jnp.float32)
    for k in range(nsample):
        sel = iota == cols[:, k:k + 1]
        nbr = _gdot3(sel, xyz0c, xyz1c, xyz2c)
        dirv = nbr - q
        vx, vy, vz = dirv[:, 0:1], dirv[:, 1:2], dirv[:, 2:3]
        inv = 1.0 / jnp.sqrt(vx * vx + vy * vy + vz * vz + 1e-12)
        dx = jnp.where(kiota == k, vx * inv, dx)
        dy = jnp.where(kiota == k, vy * inv, dy)
        dz = jnp.where(kiota == k, vz * inv, dz)
    dx_ref[0] = dx
    dy_ref[0] = dy
    dz_ref[0] = dz


def _ball_dn(queries, xT, radius, nsample, qb):
    B, S, _ = queries.shape
    V = xT.shape[2]
    body = functools.partial(_ball_dn_body, r2=radius * radius,
                             nsample=nsample, V=V)
    f32 = jnp.float32
    return _pc(
        body,
        grid=(B, S // qb),
        in_specs=[
            pl.BlockSpec((1, qb, 3), lambda b, i: (b, i, 0)),
            pl.BlockSpec((1, 3, V), lambda b, i: (b, 0, 0)),
            pl.BlockSpec((1, V, 3), lambda b, i: (b, 0, 0)),
        ],
        out_specs=[
            pl.BlockSpec((1, qb, nsample), lambda b, i: (b, i, 0)),
            pl.BlockSpec((1, qb, nsample), lambda b, i: (b, i, 0)),
            pl.BlockSpec((1, qb, nsample), lambda b, i: (b, i, 0)),
            pl.BlockSpec((1, qb, nsample), lambda b, i: (b, i, 0)),
        ],
        out_shape=[
            jax.ShapeDtypeStruct((B, S, nsample), jnp.int32),
            jax.ShapeDtypeStruct((B, S, nsample), f32),
            jax.ShapeDtypeStruct((B, S, nsample), f32),
            jax.ShapeDtypeStruct((B, S, nsample), f32),
        ],
    )(queries, xT, queries)


def _ball_body(q_ref, xT_ref, ni_ref, *, r2, nsample, V):
    q = q_ref[0]
    xT = xT_ref[0]
    d = ((q[:, 0:1] - xT[0:1, :]) ** 2
         + (q[:, 1:2] - xT[1:2, :]) ** 2
         + (q[:, 2:3] - xT[2:3, :]) ** 2)
    Qb = q.shape[0]
    iota = lax.broadcasted_iota(jnp.int32, (Qb, V), 1)
    val = jnp.where(d > r2, V, iota)
    cols = jnp.full((Qb, nsample), V, jnp.int32)
    kiota = lax.broadcasted_iota(jnp.int32, (Qb, nsample), 1)
    for k in range(nsample):
        m = jnp.min(val, axis=1, keepdims=True)
        cols = jnp.where(kiota == k, m, cols)
        val = jnp.where(val == m, V, val)
    first = cols[:, 0:1]
    ni_ref[0] = jnp.where(cols == V, first, cols)


def _ball(queries, xT, radius, nsample, qb):
    B, S, _ = queries.shape
    V = xT.shape[2]
    body = functools.partial(_ball_body, r2=radius * radius, nsample=nsample, V=V)
    return _pc(
        body,
        grid=(B, S // qb),
        in_specs=[
            pl.BlockSpec((1, qb, 3), lambda b, i: (b, i, 0)),
            pl.BlockSpec((1, 3, V), lambda b, i: (b, 0, 0)),
        ],
        out_specs=pl.BlockSpec((1, qb, nsample), lambda b, i: (b, i, 0)),
        out_shape=jax.ShapeDtypeStruct((B, S, nsample), jnp.int32),
    )(queries, xT)


def _theta(dxk, dyk, dzk, supn):
    return jnp.maximum(dxk * supn[0:1, :] + dyk * supn[1:2, :]
                       + dzk * supn[2:3, :], 0.0)


def _conv_surface_body(ni_ref, dx_ref, dy_ref, dz_ref, sup_ref, out_ref):
    sup = sup_ref[...]
    supn = sup / jnp.sqrt(jnp.sum(sup * sup, axis=0, keepdims=True) + 1e-12)
    dx, dy, dz = dx_ref[0], dy_ref[0], dz_ref[0]
    acc = None
    for k in range(NBR):
        theta = _theta(dx[:, k:k + 1], dy[:, k:k + 1], dz[:, k:k + 1], supn)
        acc = theta if acc is None else jnp.maximum(acc, theta)
    s = acc[:, 0:OUTC]
    for si in range(1, SUP):
        s = s + acc[:, si * OUTC:(si + 1) * OUTC]
    out_ref[0] = jnp.maximum(s, 0.0)


def _conv_surface(ni, dx, dy, dz, sup, qb):
    B, S, _ = ni.shape
    return _pc(
        _conv_surface_body,
        grid=(B, S // qb),
        in_specs=[
            pl.BlockSpec((1, qb, NBR), lambda b, i: (b, i, 0)),
            pl.BlockSpec((1, qb, NBR), lambda b, i: (b, i, 0)),
            pl.BlockSpec((1, qb, NBR), lambda b, i: (b, i, 0)),
            pl.BlockSpec((1, qb, NBR), lambda b, i: (b, i, 0)),
            pl.BlockSpec((3, SUP * OUTC), lambda b, i: (0, 0)),
        ],
        out_specs=pl.BlockSpec((1, qb, OUTC), lambda b, i: (b, i, 0)),
        out_shape=jax.ShapeDtypeStruct((B, S, OUTC), jnp.float32),
    )(ni, dx, dy, dz, sup)


def _conv_body(ni_ref, dx_ref, dy_ref, dz_ref, fo_ref, foblk_ref, sup_ref,
               out_ref, *, V):
    sup = sup_ref[...]
    supn = sup / jnp.sqrt(jnp.sum(sup * sup, axis=0, keepdims=True) + 1e-12)
    ni = ni_ref[0]
    dx, dy, dz = dx_ref[0], dy_ref[0], dz_ref[0]
    Qb = ni.shape[0]
    iota = lax.broadcasted_iota(jnp.int32, (Qb, V), 1)
    ft0, ft1 = _split_bf16(fo_ref[0][:, OUTC:])
    acc = None
    for k in range(NBR):
        sel = iota == ni[:, k:k + 1]
        fs = _gdot(sel, ft0, ft1)
        theta = _theta(dx[:, k:k + 1], dy[:, k:k + 1], dz[:, k:k + 1], supn)
        a = theta * fs
        acc = a if acc is None else jnp.maximum(acc, a)
    s = acc[:, 0:OUTC]
    for si in range(1, SUP):
        s = s + acc[:, si * OUTC:(si + 1) * OUTC]
    out_ref[0] = jnp.maximum(s + foblk_ref[0][:, 0:OUTC], 0.0)


def _conv(ni, dx, dy, dz, fo, sup, qb):
    B, S, _ = ni.shape
    V = fo.shape[1]
    C = fo.shape[2]
    body = functools.partial(_conv_body, V=V)
    return _pc(
        body,
        grid=(B, S // qb),
        in_specs=[
            pl.BlockSpec((1, qb, NBR), lambda b, i: (b, i, 0)),
            pl.BlockSpec((1, qb, NBR), lambda b, i: (b, i, 0)),
            pl.BlockSpec((1, qb, NBR), lambda b, i: (b, i, 0)),
            pl.BlockSpec((1, qb, NBR), lambda b, i: (b, i, 0)),
            pl.BlockSpec((1, V, C), lambda b, i: (b, 0, 0)),
            pl.BlockSpec((1, qb, C), lambda b, i: (b, i, 0)),
            pl.BlockSpec((3, SUP * OUTC), lambda b, i: (0, 0)),
        ],
        out_specs=pl.BlockSpec((1, qb, OUTC), lambda b, i: (b, i, 0)),
        out_shape=jax.ShapeDtypeStruct((B, S, OUTC), jnp.float32),
    )(ni, dx, dy, dz, fo, fo, sup)


def _hdot(a, b):
    a0, a1 = _split_bf16(a)
    b0, b1 = _split_bf16(b)
    return (jnp.dot(a0, b0, preferred_element_type=jnp.float32)
            + jnp.dot(a0, b1, preferred_element_type=jnp.float32)
            + jnp.dot(a1, b0, preferred_element_type=jnp.float32))


def _matmul_body(x_ref, w_ref, b_ref, o_ref):
    o_ref[0] = _hdot(x_ref[0], w_ref[...]) + b_ref[...]


def _matmul(x, w, b):
    B, S, Cin = x.shape
    Cout = w.shape[1]
    return _pc(
        _matmul_body,
        grid=(B,),
        in_specs=[
            pl.BlockSpec((1, S, Cin), lambda b: (b, 0, 0)),
            pl.BlockSpec((Cin, Cout), lambda b: (0, 0)),
            pl.BlockSpec((1, Cout), lambda b: (0, 0)),
        ],
        out_specs=pl.BlockSpec((1, S, Cout), lambda b: (b, 0, 0)),
        out_shape=jax.ShapeDtypeStruct((B, S, Cout), jnp.float32),
    )(x, w, b.reshape(1, Cout))


def _poolmax_body(ni_ref, feat_ref, o_ref, *, V, nsample):
    ni = ni_ref[0]
    feat = feat_ref[0]
    Sb = ni.shape[0]
    iota = lax.broadcasted_iota(jnp.int32, (Sb, V), 1)
    f0, f1 = _split_bf16(feat)
    out = None
    for k in range(nsample):
        sel = iota == ni[:, k:k + 1]
        g = _gdot(sel, f0, f1)
        out = g if out is None else jnp.maximum(out, g)
    o_ref[0] = out


def _poolmax(ni, feat, qb):
    B, S, nsample = ni.shape
    V, C = feat.shape[1], feat.shape[2]
    body = functools.partial(_poolmax_body, V=V, nsample=nsample)
    return _pc(
        body,
        grid=(B, S // qb),
        in_specs=[
            pl.BlockSpec((1, qb, nsample), lambda b, i: (b, i, 0)),
            pl.BlockSpec((1, V, C), lambda b, i: (b, 0, 0)),
        ],
        out_specs=pl.BlockSpec((1, qb, C), lambda b, i: (b, i, 0)),
        out_shape=jax.ShapeDtypeStruct((B, S, C), jnp.float32),
    )(ni, feat)


def _nearest_body(t_ref, sT_ref, o_ref, *, S):
    t = t_ref[0]
    sT = sT_ref[0]
    d = ((t[:, 0:1] - sT[0:1, :]) ** 2
         + (t[:, 1:2] - sT[1:2, :]) ** 2
         + (t[:, 2:3] - sT[2:3, :]) ** 2)
    Tb = t.shape[0]
    m = jnp.min(d, axis=1, keepdims=True)
    iota = lax.broadcasted_iota(jnp.int32, (Tb, S), 1)
    o_ref[0] = jnp.min(jnp.where(d == m, iota, S), axis=1, keepdims=True)


def _nearest(targets, sT, qb):
    B, T, _ = targets.shape
    S = sT.shape[2]
    body = functools.partial(_nearest_body, S=S)
    return _pc(
        body,
        grid=(B, T // qb),
        in_specs=[
            pl.BlockSpec((1, qb, 3), lambda b, i: (b, i, 0)),
            pl.BlockSpec((1, 3, S), lambda b, i: (b, 0, 0)),
        ],
        out_specs=pl.BlockSpec((1, qb, 1), lambda b, i: (b, i, 0)),
        out_shape=jax.ShapeDtypeStruct((B, T, 1), jnp.int32),
    )(targets, sT)


def _gather_body(idx_ref, feat_ref, o_ref, *, S):
    idx = idx_ref[0]
    feat = feat_ref[0]
    Tb = idx.shape[0]
    iota = lax.broadcasted_iota(jnp.int32, (Tb, S), 1)
    f0, f1 = _split_bf16(feat)
    sel = iota == idx
    o_ref[0] = _gdot(sel, f0, f1)


def _gather_rows(idx, feat, qb):
    B, T, _ = idx.shape
    S, C = feat.shape[1], feat.shape[2]
    body = functools.partial(_gather_body, S=S)
    return _pc(
        body,
        grid=(B, T // qb),
        in_specs=[
            pl.BlockSpec((1, qb, 1), lambda b, i: (b, i, 0)),
            pl.BlockSpec((1, S, C), lambda b, i: (b, 0, 0)),
        ],
        out_specs=pl.BlockSpec((1, qb, C), lambda b, i: (b, i, 0)),
        out_shape=jax.ShapeDtypeStruct((B, T, C), jnp.float32),
    )(idx, feat)


def _rowmax_body(x_ref, o_ref):
    o_ref[0] = jnp.max(x_ref[0], axis=0, keepdims=True)


def _rowmax(x):
    B, S, C = x.shape
    return _pc(
        _rowmax_body,
        grid=(B,),
        in_specs=[pl.BlockSpec((1, S, C), lambda b: (b, 0, 0))],
        out_specs=pl.BlockSpec((1, 1, C), lambda b: (b, 0, 0)),
        out_shape=jax.ShapeDtypeStruct((B, 1, C), jnp.float32),
    )(x)


def _mlp_body(x_ref, w1_ref, b1_ref, w2_ref, b2_ref, w3_ref, b3_ref, o_ref):
    h = jnp.maximum(_hdot(x_ref[0], w1_ref[...]) + b1_ref[...], 0.0)
    h = jnp.maximum(_hdot(h, w2_ref[...]) + b2_ref[...], 0.0)
    o_ref[0] = _hdot(h, w3_ref[...]) + b3_ref[...]


def _mlp(x, W1, B1, W2, B2, W3, B3, qb):
    B, T, C = x.shape
    H1, H2, CO = W1.shape[1], W2.shape[1], W3.shape[1]
    return _pc(
        _mlp_body,
        grid=(B, T // qb),
        in_specs=[
            pl.BlockSpec((1, qb, C), lambda b, i: (b, i, 0)),
            pl.BlockSpec((C, H1), lambda b, i: (0, 0)),
            pl.BlockSpec((1, H1), lambda b, i: (0, 0)),
            pl.BlockSpec((H1, H2), lambda b, i: (0, 0)),
            pl.BlockSpec((1, H2), lambda b, i: (0, 0)),
            pl.BlockSpec((H2, CO), lambda b, i: (0, 0)),
            pl.BlockSpec((1, CO), lambda b, i: (0, 0)),
        ],
        out_specs=pl.BlockSpec((1, qb, CO), lambda b, i: (b, i, 0)),
        out_shape=jax.ShapeDtypeStruct((B, T, CO), jnp.float32),
    )(x, W1, B1.reshape(1, H1), W2, B2.reshape(1, H2), W3, B3.reshape(1, CO))


def kernel(vertices, onehot, dir0, w1, b1, dir1, w2, b2, dir2, w3, b3, dir3,
           w4, b4, dir4, w5, b5, dir5, w6, b6, dir6, w7, b7, dir7,
           W1, B1, W2, B2, W3, B3):
    B, N, _ = vertices.shape
    xyz0 = vertices
    xT0 = jnp.transpose(xyz0, (0, 2, 1))

    ni0, dx0, dy0, dz0 = _ball_dn(xyz0, xT0, 0.25, NBR, qb=256)
    fm0 = _conv_surface(ni0, dx0, dy0, dz0, dir0, qb=256)
    fm1 = _conv(ni0, dx0, dy0, dz0, _matmul(fm0, w1, b1), dir1, qb=256)
    fm1c = jnp.concatenate([fm0, fm1], axis=2)
    fm2 = _conv(ni0, dx0, dy0, dz0, _matmul(fm1c, w2, b2), dir2, qb=256)
    fm2c = jnp.concatenate([fm1c, fm2], axis=2)

    vp1 = xyz0[:, ::4, :]
    fp1 = _poolmax(_ball(vp1, xT0, 0.25, 4, qb=512), fm2c, qb=512)
    xT1 = jnp.transpose(vp1, (0, 2, 1))
    ni1, dx1, dy1, dz1 = _ball_dn(vp1, xT1, 0.39, NBR, qb=512)
    fm3 = _conv(ni1, dx1, dy1, dz1, _matmul(fp1, w3, b3), dir3, qb=512)
    fm3c = jnp.concatenate([fp1, fm3], axis=2)
    fm4 = _conv(ni1, dx1, dy1, dz1, _matmul(fm3c, w4, b4), dir4, qb=512)
    fm4c = jnp.concatenate([fm3c, fm4], axis=2)
    fm5 = _conv(ni1, dx1, dy1, dz1, _matmul(fm4c, w5, b5), dir5, qb=512)
    fm5c = jnp.concatenate([fm4c, fm5], axis=2)

    vp2 = vp1[:, ::4, :]
    fp2 = _poolmax(_ball(vp2, xT1, 0.39, 4, qb=128), fm5c, qb=128)
    xT2 = jnp.transpose(vp2, (0, 2, 1))
    ni2, dx2, dy2, dz2 = _ball_dn(vp2, xT2, 0.63, NBR, qb=128)
    fm6 = _conv(ni2, dx2, dy2, dz2, _matmul(fp2, w6, b6), dir6, qb=128)
    fm6c = jnp.concatenate([fp2, fm6], axis=2)
    fm7 = _conv(ni2, dx2, dy2, dz2, _matmul(fm6c, w7, b7), dir7, qb=128)
    fm7c = jnp.concatenate([fm6c, fm7], axis=2)

    fglob = _rowmax(fm7c)
    np1 = _nearest(xyz0, xT1, qb=512)
    np2 = _nearest(xyz0, xT2, qb=512)
    upA = _gather_rows(np1, jnp.concatenate([fm3c, fm4c, fm5c], axis=2), qb=512)
    upB = _gather_rows(np2, jnp.concatenate([fm6c, fm7c], axis=2), qb=512)

    fuse = jnp.concatenate([
        fm0, fm1c, fm2c, upA, upB,
        jnp.broadcast_to(fglob, (B, N, fglob.shape[2])),
        jnp.broadcast_to(onehot[:, None, :], (B, N, onehot.shape[1])),
    ], axis=2)
    return _mlp(fuse, W1, B1, W2, B2, W3, B3, qb=512)


# speedup vs baseline: 7.1515x; 1.1220x over previous
import functools

import jax
import jax.numpy as jnp
from jax import lax
from jax.experimental import pallas as pl

SUP = 7
OUTC = 32
NBR = 16

_INTERPRET = False


def _pc(body, grid, in_specs, out_specs, out_shape):
    return pl.pallas_call(
        body,
        grid=grid,
        in_specs=in_specs,
        out_specs=out_specs,
        out_shape=out_shape,
        interpret=_INTERPRET,
    )


def _split_bf16(t):
    t0 = t.astype(jnp.bfloat16)
    t1 = (t - t0.astype(jnp.float32)).astype(jnp.bfloat16)
    return t0, t1


def _gdot(sel, t0, t1):
    selb = sel.astype(jnp.bfloat16)
    return (jnp.dot(selb, t0, preferred_element_type=jnp.float32)
            + jnp.dot(selb, t1, preferred_element_type=jnp.float32))


def _hdot(a, b):
    a0, a1 = _split_bf16(a)
    b0, b1 = _split_bf16(b)
    return (jnp.dot(a0, b0, preferred_element_type=jnp.float32)
            + jnp.dot(a0, b1, preferred_element_type=jnp.float32)
            + jnp.dot(a1, b0, preferred_element_type=jnp.float32))


def _xyz_chunks(xyz):
    t0 = xyz.astype(jnp.bfloat16)
    r1 = xyz - t0.astype(jnp.float32)
    t1 = r1.astype(jnp.bfloat16)
    t2 = (r1 - t1.astype(jnp.float32)).astype(jnp.bfloat16)
    return jnp.concatenate([t0, t1, t2], axis=1)


def _normsup(sup):
    return sup / jnp.sqrt(jnp.sum(sup * sup, axis=0, keepdims=True) + 1e-12)


def _theta(dxk, dyk, dzk, supn):
    return jnp.maximum(dxk * supn[0:1, :] + dyk * supn[1:2, :]
                       + dzk * supn[2:3, :], 0.0)


def _sumsup(acc):
    s = acc[:, 0:OUTC]
    for si in range(1, SUP):
        s = s + acc[:, si * OUTC:(si + 1) * OUTC]
    return s


def _ball_dn_core(q_ref, xT_ref, xyz_ref, sup_ref, out_refs,
                  r2, nsample, V):
    q = q_ref[0]
    xT = xT_ref[0]
    d = ((q[:, 0:1] - xT[0:1, :]) ** 2
         + (q[:, 1:2] - xT[1:2, :]) ** 2
         + (q[:, 2:3] - xT[2:3, :]) ** 2)
    Qb = q.shape[0]
    iota = lax.broadcasted_iota(jnp.int32, (Qb, V), 1)
    val = jnp.where(d > r2, V, iota)
    kiota = lax.broadcasted_iota(jnp.int32, (Qb, nsample), 1)
    xyzc = _xyz_chunks(xyz_ref[0])
    cols = jnp.zeros((Qb, nsample), jnp.int32)
    dx = jnp.zeros((Qb, nsample), jnp.float32)
    dy = jnp.zeros((Qb, nsample), jnp.float32)
    dz = jnp.zeros((Qb, nsample), jnp.float32)
    if sup_ref is not None:
        supn = _normsup(sup_ref[...])
        acc = None
    first = dx0 = dy0 = dz0 = None
    for k in range(nsample):
        m = jnp.min(val, axis=1, keepdims=True)
        sel = val == m
        val = jnp.where(sel, V, val)
        nbr9 = jnp.dot(sel.astype(jnp.bfloat16), xyzc,
                       preferred_element_type=jnp.float32)
        nbr = nbr9[:, 0:3] + nbr9[:, 3:6] + nbr9[:, 6:9]
        dirv = nbr - q
        vx, vy, vz = dirv[:, 0:1], dirv[:, 1:2], dirv[:, 2:3]
        inv = 1.0 / jnp.sqrt(vx * vx + vy * vy + vz * vz + 1e-12)
        vx, vy, vz = vx * inv, vy * inv, vz * inv
        if k == 0:
            first, dx0, dy0, dz0 = m, vx, vy, vz
        else:
            pad = m == V
            m = jnp.where(pad, first, m)
            vx = jnp.where(pad, dx0, vx)
            vy = jnp.where(pad, dy0, vy)
            vz = jnp.where(pad, dz0, vz)
        sk = kiota == k
        cols = jnp.where(sk, m, cols)
        dx = jnp.where(sk, vx, dx)
        dy = jnp.where(sk, vy, dy)
        dz = jnp.where(sk, vz, dz)
        if sup_ref is not None:
            th = _theta(vx, vy, vz, supn)
            acc = th if acc is None else jnp.maximum(acc, th)
    out_refs[0][0] = cols
    out_refs[1][0] = dx
    out_refs[2][0] = dy
    out_refs[3][0] = dz
    if sup_ref is not None:
        out_refs[4][0] = jnp.maximum(_sumsup(acc), 0.0)


def _ball_dn(queries, xT, radius, nsample, qb, sup=None):
    B, S, _ = queries.shape
    V = xT.shape[2]
    f32 = jnp.float32
    blk = lambda c: pl.BlockSpec((1, qb, c), lambda b, i: (b, i, 0))
    in_specs = [
        pl.BlockSpec((1, qb, 3), lambda b, i: (b, i, 0)),
        pl.BlockSpec((1, 3, V), lambda b, i: (b, 0, 0)),
        pl.BlockSpec((1, V, 3), lambda b, i: (b, 0, 0)),
    ]
    out_specs = [blk(nsample)] * 4
    out_shape = [jax.ShapeDtypeStruct((B, S, nsample), jnp.int32)] + \
                [jax.ShapeDtypeStruct((B, S, nsample), f32)] * 3
    args = [queries, xT, queries]
    if sup is not None:
        in_specs.append(pl.BlockSpec((3, SUP * OUTC), lambda b, i: (0, 0)))
        out_specs.append(blk(OUTC))
        out_shape.append(jax.ShapeDtypeStruct((B, S, OUTC), f32))
        args.append(sup)

    def body(q_ref, xT_ref, xyz_ref, *rest):
        if sup is not None:
            _ball_dn_core(q_ref, xT_ref, xyz_ref, rest[0], rest[1:],
                          radius * radius, nsample, V)
        else:
            _ball_dn_core(q_ref, xT_ref, xyz_ref, None, rest,
                          radius * radius, nsample, V)

    return _pc(body, grid=(B, S // qb), in_specs=in_specs,
               out_specs=out_specs, out_shape=out_shape)(*args)


def _agg224(ni, dx, dy, dz, t0, t1, supn, iota):
    acc = None
    for k in range(NBR):
        sel = iota == ni[:, k:k + 1]
        fs = _gdot(sel, t0, t1)
        th = _theta(dx[:, k:k + 1], dy[:, k:k + 1], dz[:, k:k + 1], supn)
        a = th * fs
        acc = a if acc is None else jnp.maximum(acc, a)
    return _sumsup(acc)


def _conv_multi(ni, dx, dy, dz, parts, w, b, sup, qb):
    B, S, _ = ni.shape
    V = parts[0].shape[1]
    widths = [p.shape[2] for p in parts]
    offs = [0]
    for c in widths:
        offs.append(offs[-1] + c)
    nparts = len(parts)

    def body(*refs):
        ni_ref, dx_ref, dy_ref, dz_ref = refs[0:4]
        pf = refs[4:4 + nparts]
        pb = refs[4 + nparts:4 + 2 * nparts]
        w_ref, b_ref, sup_ref, out_ref = refs[4 + 2 * nparts:]
        wv = w_ref[...]
        bv = b_ref[...]
        ftail = None
        fc = None
        for p in range(nparts):
            t = _hdot(pf[p][0], wv[offs[p]:offs[p + 1], OUTC:])
            c = _hdot(pb[p][0], wv[offs[p]:offs[p + 1], 0:OUTC])
            ftail = t if ftail is None else ftail + t
            fc = c if fc is None else fc + c
        ftail = ftail + bv[:, OUTC:]
        fc = fc + bv[:, 0:OUTC]
        t0, t1 = _split_bf16(ftail)
        supn = _normsup(sup_ref[...])
        Qb = ni_ref.shape[1]
        iota = lax.broadcasted_iota(jnp.int32, (Qb, V), 1)
        s = _agg224(ni_ref[0], dx_ref[0], dy_ref[0], dz_ref[0],
                    t0, t1, supn, iota)
        out_ref[0] = jnp.maximum(s + fc, 0.0)

    Cin = offs[-1]
    Cout = w.shape[1]
    blk = lambda c: pl.BlockSpec((1, qb, c), lambda b, i: (b, i, 0))
    full = lambda c: pl.BlockSpec((1, V, c), lambda b, i: (b, 0, 0))
    in_specs = ([blk(NBR)] * 4
                + [full(c) for c in widths]
                + [blk(c) for c in widths]
                + [pl.BlockSpec((Cin, Cout), lambda b, i: (0, 0)),
                   pl.BlockSpec((1, Cout), lambda b, i: (0, 0)),
                   pl.BlockSpec((3, SUP * OUTC), lambda b, i: (0, 0))])
    return _pc(
        body,
        grid=(B, S // qb),
        in_specs=in_specs,
        out_specs=blk(OUTC),
        out_shape=jax.ShapeDtypeStruct((B, S, OUTC), jnp.float32),
    )(ni, dx, dy, dz, *parts, *parts, w, b.reshape(1, Cout), sup)


def _stage_convs(ni, dx, dy, dz, feat, wbs, with_gmax=False):
    B, V, C0 = feat.shape
    n_w = len(wbs)
    f32 = jnp.float32
    blk = lambda c: pl.BlockSpec((1, V, c), lambda b: (b, 0, 0))
    in_specs = [blk(NBR)] * 4 + [blk(C0)]
    args = [ni, dx, dy, dz, feat]
    for (w, b, sup) in wbs:
        cin, cout = w.shape
        in_specs += [pl.BlockSpec((cin, cout), lambda b: (0, 0)),
                     pl.BlockSpec((1, cout), lambda b: (0, 0)),
                     pl.BlockSpec((3, SUP * OUTC), lambda b: (0, 0))]
        args += [w, b.reshape(1, cout), sup]
    out_specs = [blk(OUTC)] * n_w
    out_shape = [jax.ShapeDtypeStruct((B, V, OUTC), f32)] * n_w
    if with_gmax:
        part_widths = [C0] + [OUTC] * n_w
        out_specs += [pl.BlockSpec((1, 1, c), lambda b: (b, 0, 0))
                      for c in part_widths]
        out_shape += [jax.ShapeDtypeStruct((B, 1, c), f32)
                      for c in part_widths]

    def body(*refs):
        ni_ref, dx_ref, dy_ref, dz_ref, f_ref = refs[0:5]
        wrefs = refs[5:5 + 3 * n_w]
        out_refs = refs[5 + 3 * n_w:]
        niv = ni_ref[0]
        dxv, dyv, dzv = dx_ref[0], dy_ref[0], dz_ref[0]
        iota = lax.broadcasted_iota(jnp.int32, (V, V), 1)
        parts = [f_ref[0]]
        for li in range(n_w):
            w_ref, b_ref, sup_ref = wrefs[3 * li:3 * li + 3]
            wv, bv = w_ref[...], b_ref[...]
            supn = _normsup(sup_ref[...])
            off = 0
            ftail = None
            fc = None
            for p in parts:
                c = p.shape[1]
                t = _hdot(p, wv[off:off + c, OUTC:])
                h = _hdot(p, wv[off:off + c, 0:OUTC])
                ftail = t if ftail is None else ftail + t
                fc = h if fc is None else fc + h
                off += c
            ftail = ftail + bv[:, OUTC:]
            fc = fc + bv[:, 0:OUTC]
            t0, t1 = _split_bf16(ftail)
            s = _agg224(niv, dxv, dyv, dzv, t0, t1, supn, iota)
            fm = jnp.maximum(s + fc, 0.0)
            parts.append(fm)
            out_refs[li][0] = fm
        if with_gmax:
            for p_i, p in enumerate(parts):
                out_refs[n_w + p_i][0] = jnp.max(p, axis=0, keepdims=True)

    return _pc(body, grid=(B,), in_specs=in_specs,
               out_specs=out_specs, out_shape=out_shape)(*args)


def _pool_body(q_ref, xT_ref, feat_ref, o_ref, *, r2, nsample, V):
    q = q_ref[0]
    xT = xT_ref[0]
    d = ((q[:, 0:1] - xT[0:1, :]) ** 2
         + (q[:, 1:2] - xT[1:2, :]) ** 2
         + (q[:, 2:3] - xT[2:3, :]) ** 2)
    Qb = q.shape[0]
    iota = lax.broadcasted_iota(jnp.int32, (Qb, V), 1)
    val = jnp.where(d > r2, V, iota)
    t0, t1 = _split_bf16(feat_ref[0])
    out = None
    g0 = None
    for k in range(nsample):
        m = jnp.min(val, axis=1, keepdims=True)
        sel = val == m
        val = jnp.where(sel, V, val)
        g = _gdot(sel, t0, t1)
        if k == 0:
            g0 = g
        else:
            g = jnp.where(m == V, g0, g)
        out = g if out is None else jnp.maximum(out, g)
    o_ref[0] = out


def _pool(queries, xT, feat, radius, nsample, qb):
    B, S, _ = queries.shape
    V, C = feat.shape[1], feat.shape[2]
    body = functools.partial(_pool_body, r2=radius * radius,
                             nsample=nsample, V=V)
    return _pc(
        body,
        grid=(B, S // qb),
        in_specs=[
            pl.BlockSpec((1, qb, 3), lambda b, i: (b, i, 0)),
            pl.BlockSpec((1, 3, V), lambda b, i: (b, 0, 0)),
            pl.BlockSpec((1, V, C), lambda b, i: (b, 0, 0)),
        ],
        out_specs=pl.BlockSpec((1, qb, C), lambda b, i: (b, i, 0)),
        out_shape=jax.ShapeDtypeStruct((B, S, C), jnp.float32),
    )(queries, xT, feat)


def _upsample_body(t_ref, sT_ref, feat_ref, o_ref, *, S):
    t = t_ref[0]
    sT = sT_ref[0]
    d = ((t[:, 0:1] - sT[0:1, :]) ** 2
         + (t[:, 1:2] - sT[1:2, :]) ** 2
         + (t[:, 2:3] - sT[2:3, :]) ** 2)
    Tb = t.shape[0]
    m = jnp.min(d, axis=1, keepdims=True)
    iota = lax.broadcasted_iota(jnp.int32, (Tb, S), 1)
    idx = jnp.min(jnp.where(d == m, iota, S), axis=1, keepdims=True)
    t0, t1 = _split_bf16(feat_ref[0])
    o_ref[0] = _gdot(iota == idx, t0, t1)


def _upsample(targets, sT, feat, qb):
    B, T, _ = targets.shape
    S, C = feat.shape[1], feat.shape[2]
    body = functools.partial(_upsample_body, S=S)
    return _pc(
        body,
        grid=(B, T // qb),
        in_specs=[
            pl.BlockSpec((1, qb, 3), lambda b, i: (b, i, 0)),
            pl.BlockSpec((1, 3, S), lambda b, i: (b, 0, 0)),
            pl.BlockSpec((1, S, C), lambda b, i: (b, 0, 0)),
        ],
        out_specs=pl.BlockSpec((1, qb, C), lambda b, i: (b, i, 0)),
        out_shape=jax.ShapeDtypeStruct((B, T, C), jnp.float32),
    )(targets, sT, feat)


def _mlp_body(x_ref, w1_ref, b1_ref, w2_ref, b2_ref, w3_ref, b3_ref, o_ref):
    h = jnp.maximum(_hdot(x_ref[0], w1_ref[...]) + b1_ref[...], 0.0)
    h = jnp.maximum(_hdot(h, w2_ref[...]) + b2_ref[...], 0.0)
    o_ref[0] = _hdot(h, w3_ref[...]) + b3_ref[...]


def _mlp(x, W1, B1, W2, B2, W3, B3, qb):
    B, T, C = x.shape
    H1, H2, CO = W1.shape[1], W2.shape[1], W3.shape[1]
    return _pc(
        _mlp_body,
        grid=(B, T // qb),
        in_specs=[
            pl.BlockSpec((1, qb, C), lambda b, i: (b, i, 0)),
            pl.BlockSpec((C, H1), lambda b, i: (0, 0)),
            pl.BlockSpec((1, H1), lambda b, i: (0, 0)),
            pl.BlockSpec((H1, H2), lambda b, i: (0, 0)),
            pl.BlockSpec((1, H2), lambda b, i: (0, 0)),
            pl.BlockSpec((H2, CO), lambda b, i: (0, 0)),
            pl.BlockSpec((1, CO), lambda b, i: (0, 0)),
        ],
        out_specs=pl.BlockSpec((1, qb, CO), lambda b, i: (b, i, 0)),
        out_shape=jax.ShapeDtypeStruct((B, T, CO), jnp.float32),
    )(x, W1, B1.reshape(1, H1), W2, B2.reshape(1, H2), W3, B3.reshape(1, CO))


def kernel(vertices, onehot, dir0, w1, b1, dir1, w2, b2, dir2, w3, b3, dir3,
           w4, b4, dir4, w5, b5, dir5, w6, b6, dir6, w7, b7, dir7,
           W1, B1, W2, B2, W3, B3):
    B, N, _ = vertices.shape
    xyz0 = vertices
    xT0 = jnp.transpose(xyz0, (0, 2, 1))

    ni0, dx0, dy0, dz0, fm0 = _ball_dn(xyz0, xT0, 0.25, NBR, qb=512, sup=dir0)
    fm1 = _conv_multi(ni0, dx0, dy0, dz0, [fm0], w1, b1, dir1, qb=512)
    fm2 = _conv_multi(ni0, dx0, dy0, dz0, [fm0, fm1], w2, b2, dir2, qb=512)
    fm2c = jnp.concatenate([fm0, fm1, fm2], axis=2)

    vp1 = xyz0[:, ::4, :]
    fp1 = _pool(vp1, xT0, fm2c, 0.25, 4, qb=512)
    xT1 = jnp.transpose(vp1, (0, 2, 1))
    ni1, dx1, dy1, dz1 = _ball_dn(vp1, xT1, 0.39, NBR, qb=512)
    fm3, fm4, fm5 = _stage_convs(ni1, dx1, dy1, dz1, fp1,
                                 [(w3, b3, dir3), (w4, b4, dir4),
                                  (w5, b5, dir5)])
    fm5c = jnp.concatenate([fp1, fm3, fm4, fm5], axis=2)

    vp2 = vp1[:, ::4, :]
    fp2 = _pool(vp2, xT1, fm5c, 0.39, 4, qb=128)
    xT2 = jnp.transpose(vp2, (0, 2, 1))
    ni2, dx2, dy2, dz2 = _ball_dn(vp2, xT2, 0.63, NBR, qb=128)
    fm6, fm7, g_fp2, g6, g7 = _stage_convs(ni2, dx2, dy2, dz2, fp2,
                                           [(w6, b6, dir6), (w7, b7, dir7)],
                                           with_gmax=True)
    fm7c = jnp.concatenate([fp2, fm6, fm7], axis=2)
    fglob = jnp.concatenate([g_fp2, g6, g7], axis=2)

    catA = jnp.concatenate([fp1, fm3, fp1, fm3, fm4, fp1, fm3, fm4, fm5],
                           axis=2)
    catB = jnp.concatenate([fp2, fm6, fm7c], axis=2)
    upA = _upsample(xyz0, xT1, catA, qb=512)
    upB = _upsample(xyz0, xT2, catB, qb=512)

    fuse = jnp.concatenate([
        fm0, fm0, fm1, fm2c, upA, upB,
        jnp.broadcast_to(fglob, (B, N, fglob.shape[2])),
        jnp.broadcast_to(onehot[:, None, :], (B, N, onehot.shape[1])),
    ], axis=2)
    return _mlp(fuse, W1, B1, W2, B2, W3, B3, qb=512)


# speedup vs baseline: 8.4797x; 1.1857x over previous
import functools

import jax
import jax.numpy as jnp
from jax import lax
from jax.experimental import pallas as pl
from jax.experimental.pallas import tpu as pltpu
from jax.experimental.pallas import tpu_sc as plsc

SUP = 7
OUTC = 32
NBR = 16

_INTERPRET = False


def _pc(body, grid, in_specs, out_specs, out_shape):
    return pl.pallas_call(
        body,
        grid=grid,
        in_specs=in_specs,
        out_specs=out_specs,
        out_shape=out_shape,
        interpret=_INTERPRET,
    )


def _sc_gather(table, idx):
    info = plsc.get_sparse_core_info()
    NC, NS = info.num_cores, info.num_subcores
    NW = NC * NS
    B = idx.shape[0]
    D = table.shape[1]
    b_per_w = B // NW
    CH = 128
    nch = b_per_w // CH
    mesh = plsc.VectorSubcoreMesh(core_axis_name="c", subcore_axis_name="s")

    @functools.partial(
        pl.kernel, mesh=mesh,
        out_type=jax.ShapeDtypeStruct((B, D), jnp.float32),
        scratch_types=[
            pltpu.VMEM((CH,), jnp.int32),
            pltpu.VMEM((CH, D), jnp.float32),
            pltpu.SemaphoreType.DMA,
        ],
    )
    def k(table_hbm, idx_hbm, out_hbm, idx_v, rows_v, sem):
        wid = lax.axis_index("s") * NC + lax.axis_index("c")
        base = wid * b_per_w
        for c in range(nch):
            off = base + c * CH
            pltpu.sync_copy(idx_hbm.at[pl.ds(off, CH)], idx_v)
            pltpu.async_copy(table_hbm.at[idx_v], rows_v, sem).wait()
            pltpu.sync_copy(rows_v, out_hbm.at[pl.ds(off, CH)])

    return k(table, idx)


def _split_bf16(t):
    t0 = t.astype(jnp.bfloat16)
    t1 = (t - t0.astype(jnp.float32)).astype(jnp.bfloat16)
    return t0, t1


def _gdot(sel, t0, t1):
    selb = sel.astype(jnp.bfloat16)
    return (jnp.dot(selb, t0, preferred_element_type=jnp.float32)
            + jnp.dot(selb, t1, preferred_element_type=jnp.float32))


def _hdot(a, b):
    a0, a1 = _split_bf16(a)
    b0, b1 = _split_bf16(b)
    return (jnp.dot(a0, b0, preferred_element_type=jnp.float32)
            + jnp.dot(a0, b1, preferred_element_type=jnp.float32)
            + jnp.dot(a1, b0, preferred_element_type=jnp.float32))


def _xyz_chunks(xyz):
    t0 = xyz.astype(jnp.bfloat16)
    r1 = xyz - t0.astype(jnp.float32)
    t1 = r1.astype(jnp.bfloat16)
    t2 = (r1 - t1.astype(jnp.float32)).astype(jnp.bfloat16)
    return jnp.concatenate([t0, t1, t2], axis=1)


def _normsup(sup):
    return sup / jnp.sqrt(jnp.sum(sup * sup, axis=0, keepdims=True) + 1e-12)


def _theta(dxk, dyk, dzk, supn):
    return jnp.maximum(dxk * supn[0:1, :] + dyk * supn[1:2, :]
                       + dzk * supn[2:3, :], 0.0)


def _sumsup(acc):
    s = acc[:, 0:OUTC]
    for si in range(1, SUP):
        s = s + acc[:, si * OUTC:(si + 1) * OUTC]
    return s


def _ball_dn_core(q_ref, xT_ref, xyz_ref, sup_ref, out_refs,
                  r2, nsample, V, with_off=False):
    q = q_ref[0]
    xT = xT_ref[0]
    d = ((q[:, 0:1] - xT[0:1, :]) ** 2
         + (q[:, 1:2] - xT[1:2, :]) ** 2
         + (q[:, 2:3] - xT[2:3, :]) ** 2)
    Qb = q.shape[0]
    iota = lax.broadcasted_iota(jnp.int32, (Qb, V), 1)
    val = jnp.where(d > r2, V, iota)
    kiota = lax.broadcasted_iota(jnp.int32, (Qb, nsample), 1)
    xyzc = _xyz_chunks(xyz_ref[0])
    cols = jnp.zeros((Qb, nsample), jnp.int32)
    dx = jnp.zeros((Qb, nsample), jnp.float32)
    dy = jnp.zeros((Qb, nsample), jnp.float32)
    dz = jnp.zeros((Qb, nsample), jnp.float32)
    if sup_ref is not None:
        supn = _normsup(sup_ref[...])
        acc = None
    first = dx0 = dy0 = dz0 = None
    for k in range(nsample):
        m = jnp.min(val, axis=1, keepdims=True)
        sel = val == m
        val = jnp.where(sel, V, val)
        nbr9 = jnp.dot(sel.astype(jnp.bfloat16), xyzc,
                       preferred_element_type=jnp.float32)
        nbr = nbr9[:, 0:3] + nbr9[:, 3:6] + nbr9[:, 6:9]
        dirv = nbr - q
        vx, vy, vz = dirv[:, 0:1], dirv[:, 1:2], dirv[:, 2:3]
        inv = 1.0 / jnp.sqrt(vx * vx + vy * vy + vz * vz + 1e-12)
        vx, vy, vz = vx * inv, vy * inv, vz * inv
        if k == 0:
            first, dx0, dy0, dz0 = m, vx, vy, vz
        else:
            pad = m == V
            m = jnp.where(pad, first, m)
            vx = jnp.where(pad, dx0, vx)
            vy = jnp.where(pad, dy0, vy)
            vz = jnp.where(pad, dz0, vz)
        sk = kiota == k
        cols = jnp.where(sk, m, cols)
        dx = jnp.where(sk, vx, dx)
        dy = jnp.where(sk, vy, dy)
        dz = jnp.where(sk, vz, dz)
        if sup_ref is not None:
            th = _theta(vx, vy, vz, supn)
            acc = th if acc is None else jnp.maximum(acc, th)
    out_refs[0][0] = cols
    out_refs[1][0] = dx
    out_refs[2][0] = dy
    out_refs[3][0] = dz
    nxt = 4
    if with_off:
        out_refs[nxt][0] = cols + pl.program_id(0) * V
        nxt += 1
    if sup_ref is not None:
        out_refs[nxt][0] = jnp.maximum(_sumsup(acc), 0.0)


def _ball_dn(queries, xT, radius, nsample, qb, sup=None, with_off=False):
    B, S, _ = queries.shape
    V = xT.shape[2]
    f32 = jnp.float32
    blk = lambda c: pl.BlockSpec((1, qb, c), lambda b, i: (b, i, 0))
    in_specs = [
        pl.BlockSpec((1, qb, 3), lambda b, i: (b, i, 0)),
        pl.BlockSpec((1, 3, V), lambda b, i: (b, 0, 0)),
        pl.BlockSpec((1, V, 3), lambda b, i: (b, 0, 0)),
    ]
    out_specs = [blk(nsample)] * 4
    out_shape = [jax.ShapeDtypeStruct((B, S, nsample), jnp.int32)] + \
                [jax.ShapeDtypeStruct((B, S, nsample), f32)] * 3
    args = [queries, xT, queries]
    if with_off:
        out_specs.append(blk(nsample))
        out_shape.append(jax.ShapeDtypeStruct((B, S, nsample), jnp.int32))
    if sup is not None:
        in_specs.append(pl.BlockSpec((3, SUP * OUTC), lambda b, i: (0, 0)))
        out_specs.append(blk(OUTC))
        out_shape.append(jax.ShapeDtypeStruct((B, S, OUTC), f32))
        args.append(sup)

    def body(q_ref, xT_ref, xyz_ref, *rest):
        if sup is not None:
            _ball_dn_core(q_ref, xT_ref, xyz_ref, rest[0], rest[1:],
                          radius * radius, nsample, V, with_off=with_off)
        else:
            _ball_dn_core(q_ref, xT_ref, xyz_ref, None, rest,
                          radius * radius, nsample, V, with_off=with_off)

    return _pc(body, grid=(B, S // qb), in_specs=in_specs,
               out_specs=out_specs, out_shape=out_shape)(*args)


def _agg224(ni, dx, dy, dz, t0, t1, supn, iota):
    acc = None
    for k in range(NBR):
        sel = iota == ni[:, k:k + 1]
        fs = _gdot(sel, t0, t1)
        th = _theta(dx[:, k:k + 1], dy[:, k:k + 1], dz[:, k:k + 1], supn)
        a = th * fs
        acc = a if acc is None else jnp.maximum(acc, a)
    return _sumsup(acc)


def _fo_full(parts, w, b):
    B, V, _ = parts[0].shape
    widths = [p.shape[2] for p in parts]
    offs = [0]
    for c in widths:
        offs.append(offs[-1] + c)
    nparts = len(parts)
    Cin, Cout = offs[-1], w.shape[1]

    def body(*refs):
        prefs = refs[:nparts]
        w_ref, b_ref, o_ref = refs[nparts:]
        wv = w_ref[...]
        acc = None
        for p in range(nparts):
            t = _hdot(prefs[p][0], wv[offs[p]:offs[p + 1], :])
            acc = t if acc is None else acc + t
        o_ref[0] = acc + b_ref[...]

    full = lambda c: pl.BlockSpec((1, V, c), lambda b: (b, 0, 0))
    return _pc(
        body,
        grid=(B,),
        in_specs=[full(c) for c in widths]
                 + [pl.BlockSpec((Cin, Cout), lambda b: (0, 0)),
                    pl.BlockSpec((1, Cout), lambda b: (0, 0))],
        out_specs=full(Cout),
        out_shape=jax.ShapeDtypeStruct((B, V, Cout), jnp.float32),
    )(*parts, w, b.reshape(1, Cout))


def _conv_agg(dx, dy, dz, g, fo, sup, qb):
    B, S, _ = dx.shape
    C = fo.shape[2]

    def body(dx_ref, dy_ref, dz_ref, g_ref, fo_ref, sup_ref, out_ref):
        supn = _normsup(sup_ref[...])
        dxv, dyv, dzv = dx_ref[0], dy_ref[0], dz_ref[0]
        acc = None
        for k in range(NBR):
            fs = g_ref[k, 0][:, OUTC:]
            th = _theta(dxv[:, k:k + 1], dyv[:, k:k + 1], dzv[:, k:k + 1],
                        supn)
            a = th * fs
            acc = a if acc is None else jnp.maximum(acc, a)
        out_ref[0] = jnp.maximum(_sumsup(acc) + fo_ref[0][:, 0:OUTC], 0.0)

    blk = lambda c: pl.BlockSpec((1, qb, c), lambda b, i: (b, i, 0))
    return _pc(
        body,
        grid=(B, S // qb),
        in_specs=[blk(NBR), blk(NBR), blk(NBR),
                  pl.BlockSpec((NBR, 1, qb, C), lambda b, i: (0, b, i, 0)),
                  blk(C),
                  pl.BlockSpec((3, SUP * OUTC), lambda b, i: (0, 0))],
        out_specs=blk(OUTC),
        out_shape=jax.ShapeDtypeStruct((B, S, OUTC), jnp.float32),
    )(dx, dy, dz, g, fo, sup)


def _stage_convs(ni, dx, dy, dz, feat, wbs, with_gmax=False):
    B, V, C0 = feat.shape
    n_w = len(wbs)
    f32 = jnp.float32
    blk = lambda c: pl.BlockSpec((1, V, c), lambda b: (b, 0, 0))
    in_specs = [blk(NBR)] * 4 + [blk(C0)]
    args = [ni, dx, dy, dz, feat]
    for (w, b, sup) in wbs:
        cin, cout = w.shape
        in_specs += [pl.BlockSpec((cin, cout), lambda b: (0, 0)),
                     pl.BlockSpec((1, cout), lambda b: (0, 0)),
                     pl.BlockSpec((3, SUP * OUTC), lambda b: (0, 0))]
        args += [w, b.reshape(1, cout), sup]
    out_specs = [blk(OUTC)] * n_w
    out_shape = [jax.ShapeDtypeStruct((B, V, OUTC), f32)] * n_w
    if with_gmax:
        part_widths = [C0] + [OUTC] * n_w
        out_specs += [pl.BlockSpec((1, 1, c), lambda b: (b, 0, 0))
                      for c in part_widths]
        out_shape += [jax.ShapeDtypeStruct((B, 1, c), f32)
                      for c in part_widths]

    def body(*refs):
        ni_ref, dx_ref, dy_ref, dz_ref, f_ref = refs[0:5]
        wrefs = refs[5:5 + 3 * n_w]
        out_refs = refs[5 + 3 * n_w:]
        niv = ni_ref[0]
        dxv, dyv, dzv = dx_ref[0], dy_ref[0], dz_ref[0]
        iota = lax.broadcasted_iota(jnp.int32, (V, V), 1)
        parts = [f_ref[0]]
        for li in range(n_w):
            w_ref, b_ref, sup_ref = wrefs[3 * li:3 * li + 3]
            wv, bv = w_ref[...], b_ref[...]
            supn = _normsup(sup_ref[...])
            off = 0
            ftail = None
            fc = None
            for p in parts:
                c = p.shape[1]
                t = _hdot(p, wv[off:off + c, OUTC:])
                h = _hdot(p, wv[off:off + c, 0:OUTC])
                ftail = t if ftail is None else ftail + t
                fc = h if fc is None else fc + h
                off += c
            ftail = ftail + bv[:, OUTC:]
            fc = fc + bv[:, 0:OUTC]
            t0, t1 = _split_bf16(ftail)
            s = _agg224(niv, dxv, dyv, dzv, t0, t1, supn, iota)
            fm = jnp.maximum(s + fc, 0.0)
            parts.append(fm)
            out_refs[li][0] = fm
        if with_gmax:
            for p_i, p in enumerate(parts):
                out_refs[n_w + p_i][0] = jnp.max(p, axis=0, keepdims=True)

    return _pc(body, grid=(B,), in_specs=in_specs,
               out_specs=out_specs, out_shape=out_shape)(*args)


def _pool_body(q_ref, xT_ref, feat_ref, o_ref, *, r2, nsample, V):
    q = q_ref[0]
    xT = xT_ref[0]
    d = ((q[:, 0:1] - xT[0:1, :]) ** 2
         + (q[:, 1:2] - xT[1:2, :]) ** 2
         + (q[:, 2:3] - xT[2:3, :]) ** 2)
    Qb = q.shape[0]
    iota = lax.broadcasted_iota(jnp.int32, (Qb, V), 1)
    val = jnp.where(d > r2, V, iota)
    t0, t1 = _split_bf16(feat_ref[0])
    out = None
    g0 = None
    for k in range(nsample):
        m = jnp.min(val, axis=1, keepdims=True)
        sel = val == m
        val = jnp.where(sel, V, val)
        g = _gdot(sel, t0, t1)
        if k == 0:
            g0 = g
        else:
            g = jnp.where(m == V, g0, g)
        out = g if out is None else jnp.maximum(out, g)
    o_ref[0] = out


def _pool(queries, xT, feat, radius, nsample, qb):
    B, S, _ = queries.shape
    V, C = feat.shape[1], feat.shape[2]
    body = functools.partial(_pool_body, r2=radius * radius,
                             nsample=nsample, V=V)
    return _pc(
        body,
        grid=(B, S // qb),
        in_specs=[
            pl.BlockSpec((1, qb, 3), lambda b, i: (b, i, 0)),
            pl.BlockSpec((1, 3, V), lambda b, i: (b, 0, 0)),
            pl.BlockSpec((1, V, C), lambda b, i: (b, 0, 0)),
        ],
        out_specs=pl.BlockSpec((1, qb, C), lambda b, i: (b, i, 0)),
        out_shape=jax.ShapeDtypeStruct((B, S, C), jnp.float32),
    )(queries, xT, feat)


def _upsample_body(t_ref, sT_ref, feat_ref, o_ref, *, S):
    t = t_ref[0]
    sT = sT_ref[0]
    d = ((t[:, 0:1] - sT[0:1, :]) ** 2
         + (t[:, 1:2] - sT[1:2, :]) ** 2
         + (t[:, 2:3] - sT[2:3, :]) ** 2)
    Tb = t.shape[0]
    m = jnp.min(d, axis=1, keepdims=True)
    iota = lax.broadcasted_iota(jnp.int32, (Tb, S), 1)
    idx = jnp.min(jnp.where(d == m, iota, S), axis=1, keepdims=True)
    t0, t1 = _split_bf16(feat_ref[0])
    o_ref[0] = _gdot(iota == idx, t0, t1)


def _upsample(targets, sT, feat, qb):
    B, T, _ = targets.shape
    S, C = feat.shape[1], feat.shape[2]
    body = functools.partial(_upsample_body, S=S)
    return _pc(
        body,
        grid=(B, T // qb),
        in_specs=[
            pl.BlockSpec((1, qb, 3), lambda b, i: (b, i, 0)),
            pl.BlockSpec((1, 3, S), lambda b, i: (b, 0, 0)),
            pl.BlockSpec((1, S, C), lambda b, i: (b, 0, 0)),
        ],
        out_specs=pl.BlockSpec((1, qb, C), lambda b, i: (b, i, 0)),
        out_shape=jax.ShapeDtypeStruct((B, T, C), jnp.float32),
    )(targets, sT, feat)


def _mlp_body(x_ref, w1_ref, b1_ref, w2_ref, b2_ref, w3_ref, b3_ref, o_ref):
    h = jnp.maximum(_hdot(x_ref[0], w1_ref[...]) + b1_ref[...], 0.0)
    h = jnp.maximum(_hdot(h, w2_ref[...]) + b2_ref[...], 0.0)
    o_ref[0] = _hdot(h, w3_ref[...]) + b3_ref[...]


def _mlp(x, W1, B1, W2, B2, W3, B3, qb):
    B, T, C = x.shape
    H1, H2, CO = W1.shape[1], W2.shape[1], W3.shape[1]
    return _pc(
        _mlp_body,
        grid=(B, T // qb),
        in_specs=[
            pl.BlockSpec((1, qb, C), lambda b, i: (b, i, 0)),
            pl.BlockSpec((C, H1), lambda b, i: (0, 0)),
            pl.BlockSpec((1, H1), lambda b, i: (0, 0)),
            pl.BlockSpec((H1, H2), lambda b, i: (0, 0)),
            pl.BlockSpec((1, H2), lambda b, i: (0, 0)),
            pl.BlockSpec((H2, CO), lambda b, i: (0, 0)),
            pl.BlockSpec((1, CO), lambda b, i: (0, 0)),
        ],
        out_specs=pl.BlockSpec((1, qb, CO), lambda b, i: (b, i, 0)),
        out_shape=jax.ShapeDtypeStruct((B, T, CO), jnp.float32),
    )(x, W1, B1.reshape(1, H1), W2, B2.reshape(1, H2), W3, B3.reshape(1, CO))


def kernel(vertices, onehot, dir0, w1, b1, dir1, w2, b2, dir2, w3, b3, dir3,
           w4, b4, dir4, w5, b5, dir5, w6, b6, dir6, w7, b7, dir7,
           W1, B1, W2, B2, W3, B3):
    B, N, _ = vertices.shape
    xyz0 = vertices
    xT0 = jnp.transpose(xyz0, (0, 2, 1))

    ni0, dx0, dy0, dz0, nioff, fm0 = _ball_dn(xyz0, xT0, 0.25, NBR, qb=512,
                                              sup=dir0, with_off=True)
    idx0 = jnp.transpose(nioff, (2, 0, 1)).reshape(NBR * B * N)
    fo1 = _fo_full([fm0], w1, b1)
    g1 = _sc_gather(fo1.reshape(B * N, fo1.shape[2]), idx0)
    fm1 = _conv_agg(dx0, dy0, dz0, g1.reshape(NBR, B, N, -1), fo1, dir1,
                    qb=256)
    fo2 = _fo_full([fm0, fm1], w2, b2)
    g2 = _sc_gather(fo2.reshape(B * N, fo2.shape[2]), idx0)
    fm2 = _conv_agg(dx0, dy0, dz0, g2.reshape(NBR, B, N, -1), fo2, dir2,
                    qb=256)
    fm2c = jnp.concatenate([fm0, fm1, fm2], axis=2)

    vp1 = xyz0[:, ::4, :]
    fp1 = _pool(vp1, xT0, fm2c, 0.25, 4, qb=512)
    xT1 = jnp.transpose(vp1, (0, 2, 1))
    ni1, dx1, dy1, dz1 = _ball_dn(vp1, xT1, 0.39, NBR, qb=512)
    fm3, fm4, fm5 = _stage_convs(ni1, dx1, dy1, dz1, fp1,
                                 [(w3, b3, dir3), (w4, b4, dir4),
                                  (w5, b5, dir5)])
    fm5c = jnp.concatenate([fp1, fm3, fm4, fm5], axis=2)

    vp2 = vp1[:, ::4, :]
    fp2 = _pool(vp2, xT1, fm5c, 0.39, 4, qb=128)
    xT2 = jnp.transpose(vp2, (0, 2, 1))
    ni2, dx2, dy2, dz2 = _ball_dn(vp2, xT2, 0.63, NBR, qb=128)
    fm6, fm7, g_fp2, g6, g7 = _stage_convs(ni2, dx2, dy2, dz2, fp2,
                                           [(w6, b6, dir6), (w7, b7, dir7)],
                                           with_gmax=True)
    fm7c = jnp.concatenate([fp2, fm6, fm7], axis=2)
    fglob = jnp.concatenate([g_fp2, g6, g7], axis=2)

    catA = jnp.concatenate([fp1, fm3, fp1, fm3, fm4, fp1, fm3, fm4, fm5],
                           axis=2)
    catB = jnp.concatenate([fp2, fm6, fm7c], axis=2)
    upA = _upsample(xyz0, xT1, catA, qb=512)
    upB = _upsample(xyz0, xT2, catB, qb=512)

    fuse = jnp.concatenate([
        fm0, fm0, fm1, fm2c, upA, upB,
        jnp.broadcast_to(fglob, (B, N, fglob.shape[2])),
        jnp.broadcast_to(onehot[:, None, :], (B, N, onehot.shape[1])),
    ], axis=2)
    return _mlp(fuse, W1, B1, W2, B2, W3, B3, qb=512)


# speedup vs baseline: 8.5992x; 1.0141x over previous
import functools

import jax
import jax.numpy as jnp
from jax import lax
from jax.experimental import pallas as pl
from jax.experimental.pallas import tpu as pltpu
from jax.experimental.pallas import tpu_sc as plsc

SUP = 7
OUTC = 32
NBR = 16

_INTERPRET = False


def _pc(body, grid, in_specs, out_specs, out_shape):
    return pl.pallas_call(
        body,
        grid=grid,
        in_specs=in_specs,
        out_specs=out_specs,
        out_shape=out_shape,
        interpret=_INTERPRET,
    )


def _sc_gather(table, idx):
    info = plsc.get_sparse_core_info()
    NC, NS = info.num_cores, info.num_subcores
    NW = NC * NS
    B = idx.shape[0]
    D = table.shape[1]
    b_per_w = B // NW
    CH = 128
    nch = b_per_w // CH
    mesh = plsc.VectorSubcoreMesh(core_axis_name="c", subcore_axis_name="s")

    @functools.partial(
        pl.kernel, mesh=mesh,
        out_type=jax.ShapeDtypeStruct((B, D), jnp.float32),
        scratch_types=[
            pltpu.VMEM((nch, CH), jnp.int32),
            pltpu.VMEM((CH, D), jnp.float32),
            pltpu.VMEM((CH, D), jnp.float32),
            pltpu.SemaphoreType.DMA,
            pltpu.SemaphoreType.DMA,
            pltpu.SemaphoreType.DMA,
            pltpu.SemaphoreType.DMA,
        ],
    )
    def k(table_hbm, idx_hbm, out_hbm, idx_v, rows0, rows1, g0, g1, o0, o1):
        wid = lax.axis_index("s") * NC + lax.axis_index("c")
        base = wid * b_per_w
        pltpu.sync_copy(idx_hbm.at[pl.ds(wid * nch, nch)], idx_v)
        bufs = (rows0, rows1)
        gsems = (g0, g1)
        osems = (o0, o1)
        gh = [None, None]
        oh = [None, None]
        gh[0] = pltpu.async_copy(table_hbm.at[idx_v.at[0]], bufs[0], gsems[0])
        for c in range(nch):
            cur = c % 2
            nxt = (c + 1) % 2
            if c + 1 < nch:
                if oh[nxt] is not None:
                    oh[nxt].wait()
                gh[nxt] = pltpu.async_copy(table_hbm.at[idx_v.at[c + 1]],
                                           bufs[nxt], gsems[nxt])
            gh[cur].wait()
            oh[cur] = pltpu.async_copy(bufs[cur],
                                       out_hbm.at[pl.ds(base + c * CH, CH)],
                                       osems[cur])
        oh[0].wait()
        oh[1].wait()

    return k(table, idx.reshape(B // CH, CH))


def _split_bf16(t):
    t0 = t.astype(jnp.bfloat16)
    t1 = (t - t0.astype(jnp.float32)).astype(jnp.bfloat16)
    return t0, t1


def _gdot(sel, t0, t1):
    selb = sel.astype(jnp.bfloat16)
    return (jnp.dot(selb, t0, preferred_element_type=jnp.float32)
            + jnp.dot(selb, t1, preferred_element_type=jnp.float32))


def _hdot(a, b):
    a0, a1 = _split_bf16(a)
    b0, b1 = _split_bf16(b)
    return (jnp.dot(a0, b0, preferred_element_type=jnp.float32)
            + jnp.dot(a0, b1, preferred_element_type=jnp.float32)
            + jnp.dot(a1, b0, preferred_element_type=jnp.float32))


def _xyz_chunks(xyz):
    t0 = xyz.astype(jnp.bfloat16)
    r1 = xyz - t0.astype(jnp.float32)
    t1 = r1.astype(jnp.bfloat16)
    t2 = (r1 - t1.astype(jnp.float32)).astype(jnp.bfloat16)
    return jnp.concatenate([t0, t1, t2], axis=1)


def _normsup(sup):
    return sup / jnp.sqrt(jnp.sum(sup * sup, axis=0, keepdims=True) + 1e-12)


def _theta(dxk, dyk, dzk, supn):
    return jnp.maximum(dxk * supn[0:1, :] + dyk * supn[1:2, :]
                       + dzk * supn[2:3, :], 0.0)


def _sumsup(acc):
    s = acc[:, 0:OUTC]
    for si in range(1, SUP):
        s = s + acc[:, si * OUTC:(si + 1) * OUTC]
    return s


def _ball_dn_core(q_ref, xT_ref, xyz_ref, sup_ref, out_refs,
                  r2, nsample, V, with_off=False):
    q = q_ref[0]
    xT = xT_ref[0]
    d = ((q[:, 0:1] - xT[0:1, :]) ** 2
         + (q[:, 1:2] - xT[1:2, :]) ** 2
         + (q[:, 2:3] - xT[2:3, :]) ** 2)
    Qb = q.shape[0]
    iota = lax.broadcasted_iota(jnp.int32, (Qb, V), 1)
    val = jnp.where(d > r2, V, iota)
    kiota = lax.broadcasted_iota(jnp.int32, (Qb, nsample), 1)
    xyzc = _xyz_chunks(xyz_ref[0])
    cols = jnp.zeros((Qb, nsample), jnp.int32)
    dx = jnp.zeros((Qb, nsample), jnp.float32)
    dy = jnp.zeros((Qb, nsample), jnp.float32)
    dz = jnp.zeros((Qb, nsample), jnp.float32)
    if sup_ref is not None:
        supn = _normsup(sup_ref[...])
        acc = None
    first = dx0 = dy0 = dz0 = None
    for k in range(nsample):
        m = jnp.min(val, axis=1, keepdims=True)
        sel = val == m
        val = jnp.where(sel, V, val)
        nbr9 = jnp.dot(sel.astype(jnp.bfloat16), xyzc,
                       preferred_element_type=jnp.float32)
        nbr = nbr9[:, 0:3] + nbr9[:, 3:6] + nbr9[:, 6:9]
        dirv = nbr - q
        vx, vy, vz = dirv[:, 0:1], dirv[:, 1:2], dirv[:, 2:3]
        inv = 1.0 / jnp.sqrt(vx * vx + vy * vy + vz * vz + 1e-12)
        vx, vy, vz = vx * inv, vy * inv, vz * inv
        if k == 0:
            first, dx0, dy0, dz0 = m, vx, vy, vz
        else:
            pad = m == V
            m = jnp.where(pad, first, m)
            vx = jnp.where(pad, dx0, vx)
            vy = jnp.where(pad, dy0, vy)
            vz = jnp.where(pad, dz0, vz)
        sk = kiota == k
        cols = jnp.where(sk, m, cols)
        dx = jnp.where(sk, vx, dx)
        dy = jnp.where(sk, vy, dy)
        dz = jnp.where(sk, vz, dz)
        if sup_ref is not None:
            th = _theta(vx, vy, vz, supn)
            acc = th if acc is None else jnp.maximum(acc, th)
    out_refs[0][0] = cols
    out_refs[1][0] = dx
    out_refs[2][0] = dy
    out_refs[3][0] = dz
    nxt = 4
    if with_off:
        out_refs[nxt][0] = cols + pl.program_id(0) * V
        nxt += 1
    if sup_ref is not None:
        out_refs[nxt][0] = jnp.maximum(_sumsup(acc), 0.0)


def _ball_dn(queries, xT, radius, nsample, qb, sup=None, with_off=False):
    B, S, _ = queries.shape
    V = xT.shape[2]
    f32 = jnp.float32
    blk = lambda c: pl.BlockSpec((1, qb, c), lambda b, i: (b, i, 0))
    in_specs = [
        pl.BlockSpec((1, qb, 3), lambda b, i: (b, i, 0)),
        pl.BlockSpec((1, 3, V), lambda b, i: (b, 0, 0)),
        pl.BlockSpec((1, V, 3), lambda b, i: (b, 0, 0)),
    ]
    out_specs = [blk(nsample)] * 4
    out_shape = [jax.ShapeDtypeStruct((B, S, nsample), jnp.int32)] + \
                [jax.ShapeDtypeStruct((B, S, nsample), f32)] * 3
    args = [queries, xT, queries]
    if with_off:
        out_specs.append(blk(nsample))
        out_shape.append(jax.ShapeDtypeStruct((B, S, nsample), jnp.int32))
    if sup is not None:
        in_specs.append(pl.BlockSpec((3, SUP * OUTC), lambda b, i: (0, 0)))
        out_specs.append(blk(OUTC))
        out_shape.append(jax.ShapeDtypeStruct((B, S, OUTC), f32))
        args.append(sup)

    def body(q_ref, xT_ref, xyz_ref, *rest):
        if sup is not None:
            _ball_dn_core(q_ref, xT_ref, xyz_ref, rest[0], rest[1:],
                          radius * radius, nsample, V, with_off=with_off)
        else:
            _ball_dn_core(q_ref, xT_ref, xyz_ref, None, rest,
                          radius * radius, nsample, V, with_off=with_off)

    return _pc(body, grid=(B, S // qb), in_specs=in_specs,
               out_specs=out_specs, out_shape=out_shape)(*args)


def _agg224(ni, dx, dy, dz, t0, t1, supn, iota):
    acc = None
    for k in range(NBR):
        sel = iota == ni[:, k:k + 1]
        fs = _gdot(sel, t0, t1)
        th = _theta(dx[:, k:k + 1], dy[:, k:k + 1], dz[:, k:k + 1], supn)
        a = th * fs
        acc = a if acc is None else jnp.maximum(acc, a)
    return _sumsup(acc)


def _fo_full(parts, w, b):
    B, V, _ = parts[0].shape
    widths = [p.shape[2] for p in parts]
    offs = [0]
    for c in widths:
        offs.append(offs[-1] + c)
    nparts = len(parts)
    Cin, Cout = offs[-1], w.shape[1]

    def body(*refs):
        prefs = refs[:nparts]
        w_ref, b_ref, o_ref = refs[nparts:]
        wv = w_ref[...]
        acc = None
        for p in range(nparts):
            t = _hdot(prefs[p][0], wv[offs[p]:offs[p + 1], :])
            acc = t if acc is None else acc + t
        o_ref[0] = acc + b_ref[...]

    full = lambda c: pl.BlockSpec((1, V, c), lambda b: (b, 0, 0))
    return _pc(
        body,
        grid=(B,),
        in_specs=[full(c) for c in widths]
                 + [pl.BlockSpec((Cin, Cout), lambda b: (0, 0)),
                    pl.BlockSpec((1, Cout), lambda b: (0, 0))],
        out_specs=full(Cout),
        out_shape=jax.ShapeDtypeStruct((B, V, Cout), jnp.float32),
    )(*parts, w, b.reshape(1, Cout))


def _conv_agg(dx, dy, dz, g, fo, sup, qb):
    B, S, _ = dx.shape
    C = fo.shape[2]

    def body(dx_ref, dy_ref, dz_ref, g_ref, fo_ref, sup_ref, out_ref):
        supn = _normsup(sup_ref[...])
        dxv, dyv, dzv = dx_ref[0], dy_ref[0], dz_ref[0]
        acc = None
        for k in range(NBR):
            fs = g_ref[k, 0][:, OUTC:]
            th = _theta(dxv[:, k:k + 1], dyv[:, k:k + 1], dzv[:, k:k + 1],
                        supn)
            a = th * fs
            acc = a if acc is None else jnp.maximum(acc, a)
        out_ref[0] = jnp.maximum(_sumsup(acc) + fo_ref[0][:, 0:OUTC], 0.0)

    blk = lambda c: pl.BlockSpec((1, qb, c), lambda b, i: (b, i, 0))
    return _pc(
        body,
        grid=(B, S // qb),
        in_specs=[blk(NBR), blk(NBR), blk(NBR),
                  pl.BlockSpec((NBR, 1, qb, C), lambda b, i: (0, b, i, 0)),
                  blk(C),
                  pl.BlockSpec((3, SUP * OUTC), lambda b, i: (0, 0))],
        out_specs=blk(OUTC),
        out_shape=jax.ShapeDtypeStruct((B, S, OUTC), jnp.float32),
    )(dx, dy, dz, g, fo, sup)


def _stage_convs(ni, dx, dy, dz, feat, wbs, with_gmax=False):
    B, V, C0 = feat.shape
    n_w = len(wbs)
    f32 = jnp.float32
    blk = lambda c: pl.BlockSpec((1, V, c), lambda b: (b, 0, 0))
    in_specs = [blk(NBR)] * 4 + [blk(C0)]
    args = [ni, dx, dy, dz, feat]
    for (w, b, sup) in wbs:
        cin, cout = w.shape
        in_specs += [pl.BlockSpec((cin, cout), lambda b: (0, 0)),
                     pl.BlockSpec((1, cout), lambda b: (0, 0)),
                     pl.BlockSpec((3, SUP * OUTC), lambda b: (0, 0))]
        args += [w, b.reshape(1, cout), sup]
    out_specs = [blk(OUTC)] * n_w
    out_shape = [jax.ShapeDtypeStruct((B, V, OUTC), f32)] * n_w
    if with_gmax:
        part_widths = [C0] + [OUTC] * n_w
        out_specs += [pl.BlockSpec((1, 1, c), lambda b: (b, 0, 0))
                      for c in part_widths]
        out_shape += [jax.ShapeDtypeStruct((B, 1, c), f32)
                      for c in part_widths]

    def body(*refs):
        ni_ref, dx_ref, dy_ref, dz_ref, f_ref = refs[0:5]
        wrefs = refs[5:5 + 3 * n_w]
        out_refs = refs[5 + 3 * n_w:]
        niv = ni_ref[0]
        dxv, dyv, dzv = dx_ref[0], dy_ref[0], dz_ref[0]
        iota = lax.broadcasted_iota(jnp.int32, (V, V), 1)
        parts = [f_ref[0]]
        for li in range(n_w):
            w_ref, b_ref, sup_ref = wrefs[3 * li:3 * li + 3]
            wv, bv = w_ref[...], b_ref[...]
            supn = _normsup(sup_ref[...])
            off = 0
            ftail = None
            fc = None
            for p in parts:
                c = p.shape[1]
                t = _hdot(p, wv[off:off + c, OUTC:])
                h = _hdot(p, wv[off:off + c, 0:OUTC])
                ftail = t if ftail is None else ftail + t
                fc = h if fc is None else fc + h
                off += c
            ftail = ftail + bv[:, OUTC:]
            fc = fc + bv[:, 0:OUTC]
            t0, t1 = _split_bf16(ftail)
            s = _agg224(niv, dxv, dyv, dzv, t0, t1, supn, iota)
            fm = jnp.maximum(s + fc, 0.0)
            parts.append(fm)
            out_refs[li][0] = fm
        if with_gmax:
            for p_i, p in enumerate(parts):
                out_refs[n_w + p_i][0] = jnp.max(p, axis=0, keepdims=True)

    return _pc(body, grid=(B,), in_specs=in_specs,
               out_specs=out_specs, out_shape=out_shape)(*args)


def _pool_body(q_ref, xT_ref, feat_ref, o_ref, *, r2, nsample, V):
    q = q_ref[0]
    xT = xT_ref[0]
    d = ((q[:, 0:1] - xT[0:1, :]) ** 2
         + (q[:, 1:2] - xT[1:2, :]) ** 2
         + (q[:, 2:3] - xT[2:3, :]) ** 2)
    Qb = q.shape[0]
    iota = lax.broadcasted_iota(jnp.int32, (Qb, V), 1)
    val = jnp.where(d > r2, V, iota)
    t0, t1 = _split_bf16(feat_ref[0])
    out = None
    g0 = None
    for k in range(nsample):
        m = jnp.min(val, axis=1, keepdims=True)
        sel = val == m
        val = jnp.where(sel, V, val)
        g = _gdot(sel, t0, t1)
        if k == 0:
            g0 = g
        else:
            g = jnp.where(m == V, g0, g)
        out = g if out is None else jnp.maximum(out, g)
    o_ref[0] = out


def _pool(queries, xT, feat, radius, nsample, qb):
    B, S, _ = queries.shape
    V, C = feat.shape[1], feat.shape[2]
    body = functools.partial(_pool_body, r2=radius * radius,
                             nsample=nsample, V=V)
    return _pc(
        body,
        grid=(B, S // qb),
        in_specs=[
            pl.BlockSpec((1, qb, 3), lambda b, i: (b, i, 0)),
            pl.BlockSpec((1, 3, V), lambda b, i: (b, 0, 0)),
            pl.BlockSpec((1, V, C), lambda b, i: (b, 0, 0)),
        ],
        out_specs=pl.BlockSpec((1, qb, C), lambda b, i: (b, i, 0)),
        out_shape=jax.ShapeDtypeStruct((B, S, C), jnp.float32),
    )(queries, xT, feat)


def _upsample_body(t_ref, sT_ref, feat_ref, o_ref, *, S):
    t = t_ref[0]
    sT = sT_ref[0]
    d = ((t[:, 0:1] - sT[0:1, :]) ** 2
         + (t[:, 1:2] - sT[1:2, :]) ** 2
         + (t[:, 2:3] - sT[2:3, :]) ** 2)
    Tb = t.shape[0]
    m = jnp.min(d, axis=1, keepdims=True)
    iota = lax.broadcasted_iota(jnp.int32, (Tb, S), 1)
    idx = jnp.min(jnp.where(d == m, iota, S), axis=1, keepdims=True)
    t0, t1 = _split_bf16(feat_ref[0])
    o_ref[0] = _gdot(iota == idx, t0, t1)


def _upsample(targets, sT, feat, qb):
    B, T, _ = targets.shape
    S, C = feat.shape[1], feat.shape[2]
    body = functools.partial(_upsample_body, S=S)
    return _pc(
        body,
        grid=(B, T // qb),
        in_specs=[
            pl.BlockSpec((1, qb, 3), lambda b, i: (b, i, 0)),
            pl.BlockSpec((1, 3, S), lambda b, i: (b, 0, 0)),
            pl.BlockSpec((1, S, C), lambda b, i: (b, 0, 0)),
        ],
        out_specs=pl.BlockSpec((1, qb, C), lambda b, i: (b, i, 0)),
        out_shape=jax.ShapeDtypeStruct((B, T, C), jnp.float32),
    )(targets, sT, feat)


def _mlp_body(x_ref, w1_ref, b1_ref, w2_ref, b2_ref, w3_ref, b3_ref, o_ref):
    h = jnp.maximum(_hdot(x_ref[0], w1_ref[...]) + b1_ref[...], 0.0)
    h = jnp.maximum(_hdot(h, w2_ref[...]) + b2_ref[...], 0.0)
    o_ref[0] = _hdot(h, w3_ref[...]) + b3_ref[...]


def _mlp(x, W1, B1, W2, B2, W3, B3, qb):
    B, T, C = x.shape
    H1, H2, CO = W1.shape[1], W2.shape[1], W3.shape[1]
    return _pc(
        _mlp_body,
        grid=(B, T // qb),
        in_specs=[
            pl.BlockSpec((1, qb, C), lambda b, i: (b, i, 0)),
            pl.BlockSpec((C, H1), lambda b, i: (0, 0)),
            pl.BlockSpec((1, H1), lambda b, i: (0, 0)),
            pl.BlockSpec((H1, H2), lambda b, i: (0, 0)),
            pl.BlockSpec((1, H2), lambda b, i: (0, 0)),
            pl.BlockSpec((H2, CO), lambda b, i: (0, 0)),
            pl.BlockSpec((1, CO), lambda b, i: (0, 0)),
        ],
        out_specs=pl.BlockSpec((1, qb, CO), lambda b, i: (b, i, 0)),
        out_shape=jax.ShapeDtypeStruct((B, T, CO), jnp.float32),
    )(x, W1, B1.reshape(1, H1), W2, B2.reshape(1, H2), W3, B3.reshape(1, CO))


def kernel(vertices, onehot, dir0, w1, b1, dir1, w2, b2, dir2, w3, b3, dir3,
           w4, b4, dir4, w5, b5, dir5, w6, b6, dir6, w7, b7, dir7,
           W1, B1, W2, B2, W3, B3):
    B, N, _ = vertices.shape
    xyz0 = vertices
    xT0 = jnp.transpose(xyz0, (0, 2, 1))

    ni0, dx0, dy0, dz0, nioff, fm0 = _ball_dn(xyz0, xT0, 0.25, NBR, qb=512,
                                              sup=dir0, with_off=True)
    idx0 = jnp.transpose(nioff, (2, 0, 1)).reshape(NBR * B * N)
    fo1 = _fo_full([fm0], w1, b1)
    g1 = _sc_gather(fo1.reshape(B * N, fo1.shape[2]), idx0)
    fm1 = _conv_agg(dx0, dy0, dz0, g1.reshape(NBR, B, N, -1), fo1, dir1,
                    qb=256)
    fo2 = _fo_full([fm0, fm1], w2, b2)
    g2 = _sc_gather(fo2.reshape(B * N, fo2.shape[2]), idx0)
    fm2 = _conv_agg(dx0, dy0, dz0, g2.reshape(NBR, B, N, -1), fo2, dir2,
                    qb=256)
    fm2c = jnp.concatenate([fm0, fm1, fm2], axis=2)

    vp1 = xyz0[:, ::4, :]
    fp1 = _pool(vp1, xT0, fm2c, 0.25, 4, qb=512)
    xT1 = jnp.transpose(vp1, (0, 2, 1))
    ni1, dx1, dy1, dz1 = _ball_dn(vp1, xT1, 0.39, NBR, qb=512)
    fm3, fm4, fm5 = _stage_convs(ni1, dx1, dy1, dz1, fp1,
                                 [(w3, b3, dir3), (w4, b4, dir4),
                                  (w5, b5, dir5)])
    fm5c = jnp.concatenate([fp1, fm3, fm4, fm5], axis=2)

    vp2 = vp1[:, ::4, :]
    fp2 = _pool(vp2, xT1, fm5c, 0.39, 4, qb=128)
    xT2 = jnp.transpose(vp2, (0, 2, 1))
    ni2, dx2, dy2, dz2 = _ball_dn(vp2, xT2, 0.63, NBR, qb=128)
    fm6, fm7, g_fp2, g6, g7 = _stage_convs(ni2, dx2, dy2, dz2, fp2,
                                           [(w6, b6, dir6), (w7, b7, dir7)],
                                           with_gmax=True)
    fm7c = jnp.concatenate([fp2, fm6, fm7], axis=2)
    fglob = jnp.concatenate([g_fp2, g6, g7], axis=2)

    catA = jnp.concatenate([fp1, fm3, fp1, fm3, fm4, fp1, fm3, fm4, fm5],
                           axis=2)
    catB = jnp.concatenate([fp2, fm6, fm7c], axis=2)
    upA = _upsample(xyz0, xT1, catA, qb=512)
    upB = _upsample(xyz0, xT2, catB, qb=512)

    fuse = jnp.concatenate([
        fm0, fm0, fm1, fm2c, upA, upB,
        jnp.broadcast_to(fglob, (B, N, fglob.shape[2])),
        jnp.broadcast_to(onehot[:, None, :], (B, N, onehot.shape[1])),
    ], axis=2)
    return _mlp(fuse, W1, B1, W2, B2, W3, B3, qb=512)


# speedup vs baseline: 8.6085x; 1.0011x over previous
import functools

import jax
import jax.numpy as jnp
from jax import lax
from jax.experimental import pallas as pl
from jax.experimental.pallas import tpu as pltpu
from jax.experimental.pallas import tpu_sc as plsc

SUP = 7
OUTC = 32
NBR = 16

_INTERPRET = False


def _pc(body, grid, in_specs, out_specs, out_shape):
    return pl.pallas_call(
        body,
        grid=grid,
        in_specs=in_specs,
        out_specs=out_specs,
        out_shape=out_shape,
        interpret=_INTERPRET,
    )


def _sc_gather(table, idx):
    info = plsc.get_sparse_core_info()
    NC, NS = info.num_cores, info.num_subcores
    NW = NC * NS
    B = idx.shape[0]
    D = table.shape[1]
    b_per_w = B // NW
    CH = 128
    nch = b_per_w // CH
    mesh = plsc.VectorSubcoreMesh(core_axis_name="c", subcore_axis_name="s")

    @functools.partial(
        pl.kernel, mesh=mesh,
        out_type=jax.ShapeDtypeStruct((B, D), jnp.float32),
        scratch_types=[
            pltpu.VMEM((nch, CH), jnp.int32),
            pltpu.VMEM((CH, D), jnp.float32),
            pltpu.VMEM((CH, D), jnp.float32),
            pltpu.SemaphoreType.DMA,
            pltpu.SemaphoreType.DMA,
            pltpu.SemaphoreType.DMA,
            pltpu.SemaphoreType.DMA,
        ],
    )
    def k(table_hbm, idx_hbm, out_hbm, idx_v, rows0, rows1, g0, g1, o0, o1):
        wid = lax.axis_index("s") * NC + lax.axis_index("c")
        base = wid * b_per_w
        pltpu.sync_copy(idx_hbm.at[pl.ds(wid * nch, nch)], idx_v)
        bufs = (rows0, rows1)
        gsems = (g0, g1)
        osems = (o0, o1)
        gh = [None, None]
        oh = [None, None]
        gh[0] = pltpu.async_copy(table_hbm.at[idx_v.at[0]], bufs[0], gsems[0])
        for c in range(nch):
            cur = c % 2
            nxt = (c + 1) % 2
            if c + 1 < nch:
                if oh[nxt] is not None:
                    oh[nxt].wait()
                gh[nxt] = pltpu.async_copy(table_hbm.at[idx_v.at[c + 1]],
                                           bufs[nxt], gsems[nxt])
            gh[cur].wait()
            oh[cur] = pltpu.async_copy(bufs[cur],
                                       out_hbm.at[pl.ds(base + c * CH, CH)],
                                       osems[cur])
        oh[0].wait()
        oh[1].wait()

    return k(table, idx.reshape(B // CH, CH))


def _split_bf16(t):
    t0 = t.astype(jnp.bfloat16)
    t1 = (t - t0.astype(jnp.float32)).astype(jnp.bfloat16)
    return t0, t1


def _gdot(sel, t0, t1):
    selb = sel.astype(jnp.bfloat16)
    return (jnp.dot(selb, t0, preferred_element_type=jnp.float32)
            + jnp.dot(selb, t1, preferred_element_type=jnp.float32))


def _hdot(a, b):
    a0, a1 = _split_bf16(a)
    b0, b1 = _split_bf16(b)
    return (jnp.dot(a0, b0, preferred_element_type=jnp.float32)
            + jnp.dot(a0, b1, preferred_element_type=jnp.float32)
            + jnp.dot(a1, b0, preferred_element_type=jnp.float32))


def _xyz_chunks(xyz):
    t0 = xyz.astype(jnp.bfloat16)
    r1 = xyz - t0.astype(jnp.float32)
    t1 = r1.astype(jnp.bfloat16)
    t2 = (r1 - t1.astype(jnp.float32)).astype(jnp.bfloat16)
    return jnp.concatenate([t0, t1, t2], axis=1)


def _normsup(sup):
    return sup / jnp.sqrt(jnp.sum(sup * sup, axis=0, keepdims=True) + 1e-12)


def _theta(dxk, dyk, dzk, supn):
    return jnp.maximum(dxk * supn[0:1, :] + dyk * supn[1:2, :]
                       + dzk * supn[2:3, :], 0.0)


def _sumsup(acc):
    s = acc[:, 0:OUTC]
    for si in range(1, SUP):
        s = s + acc[:, si * OUTC:(si + 1) * OUTC]
    return s


def _ball_dn_core(q_ref, xT_ref, xyz_ref, u_ref, sup_ref, out_refs,
                  r2, nsample, V, with_off=False):
    q = q_ref[0]
    xT = xT_ref[0]
    d = ((q[:, 0:1] - xT[0:1, :]) ** 2
         + (q[:, 1:2] - xT[1:2, :]) ** 2
         + (q[:, 2:3] - xT[2:3, :]) ** 2)
    Qb = q.shape[0]
    mask = jnp.logical_not(d > r2)
    r = jnp.dot(mask.astype(jnp.bfloat16), u_ref[...],
                preferred_element_type=jnp.float32)
    cnt = r[:, V - 1:V]
    kiota = lax.broadcasted_iota(jnp.int32, (Qb, nsample), 1)
    jcol = lax.broadcasted_iota(jnp.int32, (V, 1), 0)
    hi = ((jcol // 128) * 128).astype(jnp.bfloat16)
    lo = (jcol % 128).astype(jnp.bfloat16)
    xyzc = jnp.concatenate([_xyz_chunks(xyz_ref[0]), hi, lo], axis=1)
    cols = jnp.zeros((Qb, nsample), jnp.int32)
    dx = jnp.zeros((Qb, nsample), jnp.float32)
    dy = jnp.zeros((Qb, nsample), jnp.float32)
    dz = jnp.zeros((Qb, nsample), jnp.float32)
    if sup_ref is not None:
        supn = _normsup(sup_ref[...])
        acc = None
    first = dx0 = dy0 = dz0 = None
    for k in range(nsample):
        sel = mask & (r == float(k + 1))
        g = jnp.dot(sel.astype(jnp.bfloat16), xyzc,
                    preferred_element_type=jnp.float32)
        nbr = g[:, 0:3] + g[:, 3:6] + g[:, 6:9]
        m = (g[:, 9:10] + g[:, 10:11]).astype(jnp.int32)
        dirv = nbr - q
        vx, vy, vz = dirv[:, 0:1], dirv[:, 1:2], dirv[:, 2:3]
        inv = 1.0 / jnp.sqrt(vx * vx + vy * vy + vz * vz + 1e-12)
        vx, vy, vz = vx * inv, vy * inv, vz * inv
        if k == 0:
            first, dx0, dy0, dz0 = m, vx, vy, vz
        else:
            pad = cnt < float(k + 1)
            m = jnp.where(pad, first, m)
            vx = jnp.where(pad, dx0, vx)
            vy = jnp.where(pad, dy0, vy)
            vz = jnp.where(pad, dz0, vz)
        sk = kiota == k
        cols = jnp.where(sk, m, cols)
        dx = jnp.where(sk, vx, dx)
        dy = jnp.where(sk, vy, dy)
        dz = jnp.where(sk, vz, dz)
        if sup_ref is not None:
            th = _theta(vx, vy, vz, supn)
            acc = th if acc is None else jnp.maximum(acc, th)
    out_refs[0][0] = cols
    out_refs[1][0] = dx
    out_refs[2][0] = dy
    out_refs[3][0] = dz
    nxt = 4
    if with_off:
        out_refs[nxt][0] = cols + pl.program_id(0) * V
        nxt += 1
    if sup_ref is not None:
        out_refs[nxt][0] = jnp.maximum(_sumsup(acc), 0.0)


def _ball_dn(queries, xT, radius, nsample, qb, sup=None, with_off=False):
    B, S, _ = queries.shape
    V = xT.shape[2]
    f32 = jnp.float32
    u = jnp.triu(jnp.ones((V, V), jnp.bfloat16))
    blk = lambda c: pl.BlockSpec((1, qb, c), lambda b, i: (b, i, 0))
    in_specs = [
        pl.BlockSpec((1, qb, 3), lambda b, i: (b, i, 0)),
        pl.BlockSpec((1, 3, V), lambda b, i: (b, 0, 0)),
        pl.BlockSpec((1, V, 3), lambda b, i: (b, 0, 0)),
        pl.BlockSpec((V, V), lambda b, i: (0, 0)),
    ]
    out_specs = [blk(nsample)] * 4
    out_shape = [jax.ShapeDtypeStruct((B, S, nsample), jnp.int32)] + \
                [jax.ShapeDtypeStruct((B, S, nsample), f32)] * 3
    args = [queries, xT, queries, u]
    if with_off:
        out_specs.append(blk(nsample))
        out_shape.append(jax.ShapeDtypeStruct((B, S, nsample), jnp.int32))
    if sup is not None:
        in_specs.append(pl.BlockSpec((3, SUP * OUTC), lambda b, i: (0, 0)))
        out_specs.append(blk(OUTC))
        out_shape.append(jax.ShapeDtypeStruct((B, S, OUTC), f32))
        args.append(sup)

    def body(q_ref, xT_ref, xyz_ref, u_ref, *rest):
        if sup is not None:
            _ball_dn_core(q_ref, xT_ref, xyz_ref, u_ref, rest[0], rest[1:],
                          radius * radius, nsample, V, with_off=with_off)
        else:
            _ball_dn_core(q_ref, xT_ref, xyz_ref, u_ref, None, rest,
                          radius * radius, nsample, V, with_off=with_off)

    return _pc(body, grid=(B, S // qb), in_specs=in_specs,
               out_specs=out_specs, out_shape=out_shape)(*args)


def _agg224(ni, dx, dy, dz, t0, t1, supn, iota):
    acc = None
    for k in range(NBR):
        sel = iota == ni[:, k:k + 1]
        fs = _gdot(sel, t0, t1)
        th = _theta(dx[:, k:k + 1], dy[:, k:k + 1], dz[:, k:k + 1], supn)
        a = th * fs
        acc = a if acc is None else jnp.maximum(acc, a)
    return _sumsup(acc)


def _fo_full(parts, w, b):
    B, V, _ = parts[0].shape
    widths = [p.shape[2] for p in parts]
    offs = [0]
    for c in widths:
        offs.append(offs[-1] + c)
    nparts = len(parts)
    Cin, Cout = offs[-1], w.shape[1]

    def body(*refs):
        prefs = refs[:nparts]
        w_ref, b_ref, o_ref = refs[nparts:]
        wv = w_ref[...]
        acc = None
        for p in range(nparts):
            t = _hdot(prefs[p][0], wv[offs[p]:offs[p + 1], :])
            acc = t if acc is None else acc + t
        o_ref[0] = acc + b_ref[...]

    full = lambda c: pl.BlockSpec((1, V, c), lambda b: (b, 0, 0))
    return _pc(
        body,
        grid=(B,),
        in_specs=[full(c) for c in widths]
                 + [pl.BlockSpec((Cin, Cout), lambda b: (0, 0)),
                    pl.BlockSpec((1, Cout), lambda b: (0, 0))],
        out_specs=full(Cout),
        out_shape=jax.ShapeDtypeStruct((B, V, Cout), jnp.float32),
    )(*parts, w, b.reshape(1, Cout))


def _conv_agg(dx, dy, dz, g, fo, sup, qb):
    B, S, _ = dx.shape
    C = fo.shape[2]

    def body(dx_ref, dy_ref, dz_ref, g_ref, fo_ref, sup_ref, out_ref):
        supn = _normsup(sup_ref[...])
        dxv, dyv, dzv = dx_ref[0], dy_ref[0], dz_ref[0]
        acc = None
        for k in range(NBR):
            fs = g_ref[k, 0][:, OUTC:]
            th = _theta(dxv[:, k:k + 1], dyv[:, k:k + 1], dzv[:, k:k + 1],
                        supn)
            a = th * fs
            acc = a if acc is None else jnp.maximum(acc, a)
        out_ref[0] = jnp.maximum(_sumsup(acc) + fo_ref[0][:, 0:OUTC], 0.0)

    blk = lambda c: pl.BlockSpec((1, qb, c), lambda b, i: (b, i, 0))
    return _pc(
        body,
        grid=(B, S // qb),
        in_specs=[blk(NBR), blk(NBR), blk(NBR),
                  pl.BlockSpec((NBR, 1, qb, C), lambda b, i: (0, b, i, 0)),
                  blk(C),
                  pl.BlockSpec((3, SUP * OUTC), lambda b, i: (0, 0))],
        out_specs=blk(OUTC),
        out_shape=jax.ShapeDtypeStruct((B, S, OUTC), jnp.float32),
    )(dx, dy, dz, g, fo, sup)


def _stage_convs(ni, dx, dy, dz, feat, wbs, with_gmax=False):
    B, V, C0 = feat.shape
    n_w = len(wbs)
    f32 = jnp.float32
    blk = lambda c: pl.BlockSpec((1, V, c), lambda b: (b, 0, 0))
    in_specs = [blk(NBR)] * 4 + [blk(C0)]
    args = [ni, dx, dy, dz, feat]
    for (w, b, sup) in wbs:
        cin, cout = w.shape
        in_specs += [pl.BlockSpec((cin, cout), lambda b: (0, 0)),
                     pl.BlockSpec((1, cout), lambda b: (0, 0)),
                     pl.BlockSpec((3, SUP * OUTC), lambda b: (0, 0))]
        args += [w, b.reshape(1, cout), sup]
    out_specs = [blk(OUTC)] * n_w
    out_shape = [jax.ShapeDtypeStruct((B, V, OUTC), f32)] * n_w
    if with_gmax:
        part_widths = [C0] + [OUTC] * n_w
        out_specs += [pl.BlockSpec((1, 1, c), lambda b: (b, 0, 0))
                      for c in part_widths]
        out_shape += [jax.ShapeDtypeStruct((B, 1, c), f32)
                      for c in part_widths]

    def body(*refs):
        ni_ref, dx_ref, dy_ref, dz_ref, f_ref = refs[0:5]
        wrefs = refs[5:5 + 3 * n_w]
        out_refs = refs[5 + 3 * n_w:]
        niv = ni_ref[0]
        dxv, dyv, dzv = dx_ref[0], dy_ref[0], dz_ref[0]
        iota = lax.broadcasted_iota(jnp.int32, (V, V), 1)
        parts = [f_ref[0]]
        for li in range(n_w):
            w_ref, b_ref, sup_ref = wrefs[3 * li:3 * li + 3]
            wv, bv = w_ref[...], b_ref[...]
            supn = _normsup(sup_ref[...])
            off = 0
            ftail = None
            fc = None
            for p in parts:
                c = p.shape[1]
                t = _hdot(p, wv[off:off + c, OUTC:])
                h = _hdot(p, wv[off:off + c, 0:OUTC])
                ftail = t if ftail is None else ftail + t
                fc = h if fc is None else fc + h
                off += c
            ftail = ftail + bv[:, OUTC:]
            fc = fc + bv[:, 0:OUTC]
            t0, t1 = _split_bf16(ftail)
            s = _agg224(niv, dxv, dyv, dzv, t0, t1, supn, iota)
            fm = jnp.maximum(s + fc, 0.0)
            parts.append(fm)
            out_refs[li][0] = fm
        if with_gmax:
            for p_i, p in enumerate(parts):
                out_refs[n_w + p_i][0] = jnp.max(p, axis=0, keepdims=True)

    return _pc(body, grid=(B,), in_specs=in_specs,
               out_specs=out_specs, out_shape=out_shape)(*args)


def _pool_body(q_ref, xT_ref, feat_ref, o_ref, *, r2, nsample, V):
    q = q_ref[0]
    xT = xT_ref[0]
    d = ((q[:, 0:1] - xT[0:1, :]) ** 2
         + (q[:, 1:2] - xT[1:2, :]) ** 2
         + (q[:, 2:3] - xT[2:3, :]) ** 2)
    Qb = q.shape[0]
    iota = lax.broadcasted_iota(jnp.int32, (Qb, V), 1)
    val = jnp.where(d > r2, V, iota)
    t0, t1 = _split_bf16(feat_ref[0])
    out = None
    g0 = None
    for k in range(nsample):
        m = jnp.min(val, axis=1, keepdims=True)
        sel = val == m
        val = jnp.where(sel, V, val)
        g = _gdot(sel, t0, t1)
        if k == 0:
            g0 = g
        else:
            g = jnp.where(m == V, g0, g)
        out = g if out is None else jnp.maximum(out, g)
    o_ref[0] = out


def _pool(queries, xT, feat, radius, nsample, qb):
    B, S, _ = queries.shape
    V, C = feat.shape[1], feat.shape[2]
    body = functools.partial(_pool_body, r2=radius * radius,
                             nsample=nsample, V=V)
    return _pc(
        body,
        grid=(B, S // qb),
        in_specs=[
            pl.BlockSpec((1, qb, 3), lambda b, i: (b, i, 0)),
            pl.BlockSpec((1, 3, V), lambda b, i: (b, 0, 0)),
            pl.BlockSpec((1, V, C), lambda b, i: (b, 0, 0)),
        ],
        out_specs=pl.BlockSpec((1, qb, C), lambda b, i: (b, i, 0)),
        out_shape=jax.ShapeDtypeStruct((B, S, C), jnp.float32),
    )(queries, xT, feat)


def _upsample_body(t_ref, sT_ref, feat_ref, o_ref, *, S):
    t = t_ref[0]
    sT = sT_ref[0]
    d = ((t[:, 0:1] - sT[0:1, :]) ** 2
         + (t[:, 1:2] - sT[1:2, :]) ** 2
         + (t[:, 2:3] - sT[2:3, :]) ** 2)
    Tb = t.shape[0]
    m = jnp.min(d, axis=1, keepdims=True)
    iota = lax.broadcasted_iota(jnp.int32, (Tb, S), 1)
    idx = jnp.min(jnp.where(d == m, iota, S), axis=1, keepdims=True)
    t0, t1 = _split_bf16(feat_ref[0])
    o_ref[0] = _gdot(iota == idx, t0, t1)


def _upsample(targets, sT, feat, qb):
    B, T, _ = targets.shape
    S, C = feat.shape[1], feat.shape[2]
    body = functools.partial(_upsample_body, S=S)
    return _pc(
        body,
        grid=(B, T // qb),
        in_specs=[
            pl.BlockSpec((1, qb, 3), lambda b, i: (b, i, 0)),
            pl.BlockSpec((1, 3, S), lambda b, i: (b, 0, 0)),
            pl.BlockSpec((1, S, C), lambda b, i: (b, 0, 0)),
        ],
        out_specs=pl.BlockSpec((1, qb, C), lambda b, i: (b, i, 0)),
        out_shape=jax.ShapeDtypeStruct((B, T, C), jnp.float32),
    )(targets, sT, feat)


def _mlp_body(x_ref, w1_ref, b1_ref, w2_ref, b2_ref, w3_ref, b3_ref, o_ref):
    h = jnp.maximum(_hdot(x_ref[0], w1_ref[...]) + b1_ref[...], 0.0)
    h = jnp.maximum(_hdot(h, w2_ref[...]) + b2_ref[...], 0.0)
    o_ref[0] = _hdot(h, w3_ref[...]) + b3_ref[...]


def _mlp(x, W1, B1, W2, B2, W3, B3, qb):
    B, T, C = x.shape
    H1, H2, CO = W1.shape[1], W2.shape[1], W3.shape[1]
    return _pc(
        _mlp_body,
        grid=(B, T // qb),
        in_specs=[
            pl.BlockSpec((1, qb, C), lambda b, i: (b, i, 0)),
            pl.BlockSpec((C, H1), lambda b, i: (0, 0)),
            pl.BlockSpec((1, H1), lambda b, i: (0, 0)),
            pl.BlockSpec((H1, H2), lambda b, i: (0, 0)),
            pl.BlockSpec((1, H2), lambda b, i: (0, 0)),
            pl.BlockSpec((H2, CO), lambda b, i: (0, 0)),
            pl.BlockSpec((1, CO), lambda b, i: (0, 0)),
        ],
        out_specs=pl.BlockSpec((1, qb, CO), lambda b, i: (b, i, 0)),
        out_shape=jax.ShapeDtypeStruct((B, T, CO), jnp.float32),
    )(x, W1, B1.reshape(1, H1), W2, B2.reshape(1, H2), W3, B3.reshape(1, CO))


def kernel(vertices, onehot, dir0, w1, b1, dir1, w2, b2, dir2, w3, b3, dir3,
           w4, b4, dir4, w5, b5, dir5, w6, b6, dir6, w7, b7, dir7,
           W1, B1, W2, B2, W3, B3):
    B, N, _ = vertices.shape
    xyz0 = vertices
    xT0 = jnp.transpose(xyz0, (0, 2, 1))

    ni0, dx0, dy0, dz0, nioff, fm0 = _ball_dn(xyz0, xT0, 0.25, NBR, qb=512,
                                              sup=dir0, with_off=True)
    idx0 = jnp.transpose(nioff, (2, 0, 1)).reshape(NBR * B * N)
    fo1 = _fo_full([fm0], w1, b1)
    g1 = _sc_gather(fo1.reshape(B * N, fo1.shape[2]), idx0)
    fm1 = _conv_agg(dx0, dy0, dz0, g1.reshape(NBR, B, N, -1), fo1, dir1,
                    qb=256)
    fo2 = _fo_full([fm0, fm1], w2, b2)
    g2 = _sc_gather(fo2.reshape(B * N, fo2.shape[2]), idx0)
    fm2 = _conv_agg(dx0, dy0, dz0, g2.reshape(NBR, B, N, -1), fo2, dir2,
                    qb=256)
    fm2c = jnp.concatenate([fm0, fm1, fm2], axis=2)

    vp1 = xyz0[:, ::4, :]
    fp1 = _pool(vp1, xT0, fm2c, 0.25, 4, qb=512)
    xT1 = jnp.transpose(vp1, (0, 2, 1))
    ni1, dx1, dy1, dz1 = _ball_dn(vp1, xT1, 0.39, NBR, qb=512)
    fm3, fm4, fm5 = _stage_convs(ni1, dx1, dy1, dz1, fp1,
                                 [(w3, b3, dir3), (w4, b4, dir4),
                                  (w5, b5, dir5)])
    fm5c = jnp.concatenate([fp1, fm3, fm4, fm5], axis=2)

    vp2 = vp1[:, ::4, :]
    fp2 = _pool(vp2, xT1, fm5c, 0.39, 4, qb=128)
    xT2 = jnp.transpose(vp2, (0, 2, 1))
    ni2, dx2, dy2, dz2 = _ball_dn(vp2, xT2, 0.63, NBR, qb=128)
    fm6, fm7, g_fp2, g6, g7 = _stage_convs(ni2, dx2, dy2, dz2, fp2,
                                           [(w6, b6, dir6), (w7, b7, dir7)],
                                           with_gmax=True)
    fm7c = jnp.concatenate([fp2, fm6, fm7], axis=2)
    fglob = jnp.concatenate([g_fp2, g6, g7], axis=2)

    catA = jnp.concatenate([fp1, fm3, fp1, fm3, fm4, fp1, fm3, fm4, fm5],
                           axis=2)
    catB = jnp.concatenate([fp2, fm6, fm7c], axis=2)
    upA = _upsample(xyz0, xT1, catA, qb=512)
    upB = _upsample(xyz0, xT2, catB, qb=512)

    fuse = jnp.concatenate([
        fm0, fm0, fm1, fm2c, upA, upB,
        jnp.broadcast_to(fglob, (B, N, fglob.shape[2])),
        jnp.broadcast_to(onehot[:, None, :], (B, N, onehot.shape[1])),
    ], axis=2)
    return _mlp(fuse, W1, B1, W2, B2, W3, B3, qb=512)


# speedup vs baseline: 8.6133x; 1.0006x over previous
import functools

import jax
import jax.numpy as jnp
from jax import lax
from jax.experimental import pallas as pl
from jax.experimental.pallas import tpu as pltpu
from jax.experimental.pallas import tpu_sc as plsc

SUP = 7
OUTC = 32
NBR = 16

def _pc(body, grid, in_specs, out_specs, out_shape):
    return pl.pallas_call(
        body,
        grid=grid,
        in_specs=in_specs,
        out_specs=out_specs,
        out_shape=out_shape,
    )


def _sc_gather(table, idx):
    info = plsc.get_sparse_core_info()
    NC, NS = info.num_cores, info.num_subcores
    NW = NC * NS
    B = idx.shape[0]
    D = table.shape[1]
    b_per_w = B // NW
    CH = 128
    nch = b_per_w // CH
    mesh = plsc.VectorSubcoreMesh(core_axis_name="c", subcore_axis_name="s")

    @functools.partial(
        pl.kernel, mesh=mesh,
        out_type=jax.ShapeDtypeStruct((B, D), jnp.float32),
        scratch_types=[
            pltpu.VMEM((nch, CH), jnp.int32),
            pltpu.VMEM((CH, D), jnp.float32),
            pltpu.VMEM((CH, D), jnp.float32),
            pltpu.SemaphoreType.DMA,
            pltpu.SemaphoreType.DMA,
            pltpu.SemaphoreType.DMA,
            pltpu.SemaphoreType.DMA,
        ],
    )
    def k(table_hbm, idx_hbm, out_hbm, idx_v, rows0, rows1, g0, g1, o0, o1):
        wid = lax.axis_index("s") * NC + lax.axis_index("c")
        base = wid * b_per_w
        pltpu.sync_copy(idx_hbm.at[pl.ds(wid * nch, nch)], idx_v)
        bufs = (rows0, rows1)
        gsems = (g0, g1)
        osems = (o0, o1)
        gh = [None, None]
        oh = [None, None]
        gh[0] = pltpu.async_copy(table_hbm.at[idx_v.at[0]], bufs[0], gsems[0])
        for c in range(nch):
            cur = c % 2
            nxt = (c + 1) % 2
            if c + 1 < nch:
                if oh[nxt] is not None:
                    oh[nxt].wait()
                gh[nxt] = pltpu.async_copy(table_hbm.at[idx_v.at[c + 1]],
                                           bufs[nxt], gsems[nxt])
            gh[cur].wait()
            oh[cur] = pltpu.async_copy(bufs[cur],
                                       out_hbm.at[pl.ds(base + c * CH, CH)],
                                       osems[cur])
        oh[0].wait()
        oh[1].wait()

    return k(table, idx.reshape(B // CH, CH))


def _split_bf16(t):
    t0 = t.astype(jnp.bfloat16)
    t1 = (t - t0.astype(jnp.float32)).astype(jnp.bfloat16)
    return t0, t1


def _gdot(sel, t0, t1):
    selb = sel.astype(jnp.bfloat16)
    return (jnp.dot(selb, t0, preferred_element_type=jnp.float32)
            + jnp.dot(selb, t1, preferred_element_type=jnp.float32))


def _hdot(a, b):
    a0, a1 = _split_bf16(a)
    b0, b1 = _split_bf16(b)
    return (jnp.dot(a0, b0, preferred_element_type=jnp.float32)
            + jnp.dot(a0, b1, preferred_element_type=jnp.float32)
            + jnp.dot(a1, b0, preferred_element_type=jnp.float32))


def _xyz_chunks(xyz):
    t0 = xyz.astype(jnp.bfloat16)
    r1 = xyz - t0.astype(jnp.float32)
    t1 = r1.astype(jnp.bfloat16)
    t2 = (r1 - t1.astype(jnp.float32)).astype(jnp.bfloat16)
    return jnp.concatenate([t0, t1, t2], axis=1)


def _normsup(sup):
    return sup / jnp.sqrt(jnp.sum(sup * sup, axis=0, keepdims=True) + 1e-12)


def _theta(dxk, dyk, dzk, supn):
    return jnp.maximum(dxk * supn[0:1, :] + dyk * supn[1:2, :]
                       + dzk * supn[2:3, :], 0.0)


def _sumsup(acc):
    s = acc[:, 0:OUTC]
    for si in range(1, SUP):
        s = s + acc[:, si * OUTC:(si + 1) * OUTC]
    return s


def _ball_dn_core(q_ref, xT_ref, xyz_ref, u_ref, sup_ref, out_refs,
                  r2, nsample, V, with_off=False):
    q = q_ref[0]
    xT = xT_ref[0]
    d = ((q[:, 0:1] - xT[0:1, :]) ** 2
         + (q[:, 1:2] - xT[1:2, :]) ** 2
         + (q[:, 2:3] - xT[2:3, :]) ** 2)
    Qb = q.shape[0]
    mask = jnp.logical_not(d > r2)
    r = jnp.dot(mask.astype(jnp.bfloat16), u_ref[...],
                preferred_element_type=jnp.float32)
    cnt = r[:, V - 1:V]
    kiota = lax.broadcasted_iota(jnp.int32, (Qb, nsample), 1)
    jcol = lax.broadcasted_iota(jnp.int32, (V, 1), 0)
    hi = ((jcol // 128) * 128).astype(jnp.bfloat16)
    lo = (jcol % 128).astype(jnp.bfloat16)
    xyzc = jnp.concatenate([_xyz_chunks(xyz_ref[0]), hi, lo], axis=1)
    cols = jnp.zeros((Qb, nsample), jnp.int32)
    dx = jnp.zeros((Qb, nsample), jnp.float32)
    dy = jnp.zeros((Qb, nsample), jnp.float32)
    dz = jnp.zeros((Qb, nsample), jnp.float32)
    if sup_ref is not None:
        supn = _normsup(sup_ref[...])
        acc = None
    first = dx0 = dy0 = dz0 = None
    for k in range(nsample):
        sel = mask & (r == float(k + 1))
        g = jnp.dot(sel.astype(jnp.bfloat16), xyzc,
                    preferred_element_type=jnp.float32)
        nbr = g[:, 0:3] + g[:, 3:6] + g[:, 6:9]
        m = (g[:, 9:10] + g[:, 10:11]).astype(jnp.int32)
        dirv = nbr - q
        vx, vy, vz = dirv[:, 0:1], dirv[:, 1:2], dirv[:, 2:3]
        inv = 1.0 / jnp.sqrt(vx * vx + vy * vy + vz * vz + 1e-12)
        vx, vy, vz = vx * inv, vy * inv, vz * inv
        if k == 0:
            first, dx0, dy0, dz0 = m, vx, vy, vz
        else:
            pad = cnt < float(k + 1)
            m = jnp.where(pad, first, m)
            vx = jnp.where(pad, dx0, vx)
            vy = jnp.where(pad, dy0, vy)
            vz = jnp.where(pad, dz0, vz)
        sk = kiota == k
        cols = jnp.where(sk, m, cols)
        dx = jnp.where(sk, vx, dx)
        dy = jnp.where(sk, vy, dy)
        dz = jnp.where(sk, vz, dz)
        if sup_ref is not None:
            th = _theta(vx, vy, vz, supn)
            acc = th if acc is None else jnp.maximum(acc, th)
    out_refs[0][0] = cols
    out_refs[1][0] = dx
    out_refs[2][0] = dy
    out_refs[3][0] = dz
    nxt = 4
    if with_off:
        out_refs[nxt][0] = cols + pl.program_id(0) * V
        nxt += 1
    if sup_ref is not None:
        out_refs[nxt][0] = jnp.maximum(_sumsup(acc), 0.0)


def _ball_dn(queries, xT, radius, nsample, qb, sup=None, with_off=False):
    B, S, _ = queries.shape
    V = xT.shape[2]
    f32 = jnp.float32
    u = jnp.triu(jnp.ones((V, V), jnp.bfloat16))
    blk = lambda c: pl.BlockSpec((1, qb, c), lambda b, i: (b, i, 0))
    in_specs = [
        pl.BlockSpec((1, qb, 3), lambda b, i: (b, i, 0)),
        pl.BlockSpec((1, 3, V), lambda b, i: (b, 0, 0)),
        pl.BlockSpec((1, V, 3), lambda b, i: (b, 0, 0)),
        pl.BlockSpec((V, V), lambda b, i: (0, 0)),
    ]
    out_specs = [blk(nsample)] * 4
    out_shape = [jax.ShapeDtypeStruct((B, S, nsample), jnp.int32)] + \
                [jax.ShapeDtypeStruct((B, S, nsample), f32)] * 3
    args = [queries, xT, queries, u]
    if with_off:
        out_specs.append(blk(nsample))
        out_shape.append(jax.ShapeDtypeStruct((B, S, nsample), jnp.int32))
    if sup is not None:
        in_specs.append(pl.BlockSpec((3, SUP * OUTC), lambda b, i: (0, 0)))
        out_specs.append(blk(OUTC))
        out_shape.append(jax.ShapeDtypeStruct((B, S, OUTC), f32))
        args.append(sup)

    def body(q_ref, xT_ref, xyz_ref, u_ref, *rest):
        if sup is not None:
            _ball_dn_core(q_ref, xT_ref, xyz_ref, u_ref, rest[0], rest[1:],
                          radius * radius, nsample, V, with_off=with_off)
        else:
            _ball_dn_core(q_ref, xT_ref, xyz_ref, u_ref, None, rest,
                          radius * radius, nsample, V, with_off=with_off)

    return _pc(body, grid=(B, S // qb), in_specs=in_specs,
               out_specs=out_specs, out_shape=out_shape)(*args)


def _agg224(ni, dx, dy, dz, t0, t1, supn, iota):
    acc = None
    for k in range(NBR):
        sel = iota == ni[:, k:k + 1]
        fs = _gdot(sel, t0, t1)
        th = _theta(dx[:, k:k + 1], dy[:, k:k + 1], dz[:, k:k + 1], supn)
        a = th * fs
        acc = a if acc is None else jnp.maximum(acc, a)
    return _sumsup(acc)


def _fo_full(parts, w, b):
    B, V, _ = parts[0].shape
    widths = [p.shape[2] for p in parts]
    offs = [0]
    for c in widths:
        offs.append(offs[-1] + c)
    nparts = len(parts)
    Cin, Cout = offs[-1], w.shape[1]

    def body(*refs):
        prefs = refs[:nparts]
        w_ref, b_ref, o_ref = refs[nparts:]
        wv = w_ref[...]
        acc = None
        for p in range(nparts):
            t = _hdot(prefs[p][0], wv[offs[p]:offs[p + 1], :])
            acc = t if acc is None else acc + t
        o_ref[0] = acc + b_ref[...]

    full = lambda c: pl.BlockSpec((1, V, c), lambda b: (b, 0, 0))
    return _pc(
        body,
        grid=(B,),
        in_specs=[full(c) for c in widths]
                 + [pl.BlockSpec((Cin, Cout), lambda b: (0, 0)),
                    pl.BlockSpec((1, Cout), lambda b: (0, 0))],
        out_specs=full(Cout),
        out_shape=jax.ShapeDtypeStruct((B, V, Cout), jnp.float32),
    )(*parts, w, b.reshape(1, Cout))


def _conv_agg(dx, dy, dz, g, fo, sup, qb):
    B, S, _ = dx.shape
    C = fo.shape[2]

    def body(dx_ref, dy_ref, dz_ref, g_ref, fo_ref, sup_ref, out_ref):
        supn = _normsup(sup_ref[...])
        dxv, dyv, dzv = dx_ref[0], dy_ref[0], dz_ref[0]
        acc = None
        for k in range(NBR):
            fs = g_ref[k, 0][:, OUTC:]
            th = _theta(dxv[:, k:k + 1], dyv[:, k:k + 1], dzv[:, k:k + 1],
                        supn)
            a = th * fs
            acc = a if acc is None else jnp.maximum(acc, a)
        out_ref[0] = jnp.maximum(_sumsup(acc) + fo_ref[0][:, 0:OUTC], 0.0)

    blk = lambda c: pl.BlockSpec((1, qb, c), lambda b, i: (b, i, 0))
    return _pc(
        body,
        grid=(B, S // qb),
        in_specs=[blk(NBR), blk(NBR), blk(NBR),
                  pl.BlockSpec((NBR, 1, qb, C), lambda b, i: (0, b, i, 0)),
                  blk(C),
                  pl.BlockSpec((3, SUP * OUTC), lambda b, i: (0, 0))],
        out_specs=blk(OUTC),
        out_shape=jax.ShapeDtypeStruct((B, S, OUTC), jnp.float32),
    )(dx, dy, dz, g, fo, sup)


def _stage_convs(ni, dx, dy, dz, feat, wbs, with_gmax=False):
    B, V, C0 = feat.shape
    n_w = len(wbs)
    f32 = jnp.float32
    blk = lambda c: pl.BlockSpec((1, V, c), lambda b: (b, 0, 0))
    in_specs = [blk(NBR)] * 4 + [blk(C0)]
    args = [ni, dx, dy, dz, feat]
    for (w, b, sup) in wbs:
        cin, cout = w.shape
        in_specs += [pl.BlockSpec((cin, cout), lambda b: (0, 0)),
                     pl.BlockSpec((1, cout), lambda b: (0, 0)),
                     pl.BlockSpec((3, SUP * OUTC), lambda b: (0, 0))]
        args += [w, b.reshape(1, cout), sup]
    out_specs = [blk(OUTC)] * n_w
    out_shape = [jax.ShapeDtypeStruct((B, V, OUTC), f32)] * n_w
    if with_gmax:
        part_widths = [C0] + [OUTC] * n_w
        out_specs += [pl.BlockSpec((1, 1, c), lambda b: (b, 0, 0))
                      for c in part_widths]
        out_shape += [jax.ShapeDtypeStruct((B, 1, c), f32)
                      for c in part_widths]

    def body(*refs):
        ni_ref, dx_ref, dy_ref, dz_ref, f_ref = refs[0:5]
        wrefs = refs[5:5 + 3 * n_w]
        out_refs = refs[5 + 3 * n_w:]
        niv = ni_ref[0]
        dxv, dyv, dzv = dx_ref[0], dy_ref[0], dz_ref[0]
        iota = lax.broadcasted_iota(jnp.int32, (V, V), 1)
        parts = [f_ref[0]]
        for li in range(n_w):
            w_ref, b_ref, sup_ref = wrefs[3 * li:3 * li + 3]
            wv, bv = w_ref[...], b_ref[...]
            supn = _normsup(sup_ref[...])
            off = 0
            ftail = None
            fc = None
            for p in parts:
                c = p.shape[1]
                t = _hdot(p, wv[off:off + c, OUTC:])
                h = _hdot(p, wv[off:off + c, 0:OUTC])
                ftail = t if ftail is None else ftail + t
                fc = h if fc is None else fc + h
                off += c
            ftail = ftail + bv[:, OUTC:]
            fc = fc + bv[:, 0:OUTC]
            t0, t1 = _split_bf16(ftail)
            s = _agg224(niv, dxv, dyv, dzv, t0, t1, supn, iota)
            fm = jnp.maximum(s + fc, 0.0)
            parts.append(fm)
            out_refs[li][0] = fm
        if with_gmax:
            for p_i, p in enumerate(parts):
                out_refs[n_w + p_i][0] = jnp.max(p, axis=0, keepdims=True)

    return _pc(body, grid=(B,), in_specs=in_specs,
               out_specs=out_specs, out_shape=out_shape)(*args)


def _pool_body(q_ref, xT_ref, feat_ref, o_ref, *, r2, nsample, V):
    q = q_ref[0]
    xT = xT_ref[0]
    d = ((q[:, 0:1] - xT[0:1, :]) ** 2
         + (q[:, 1:2] - xT[1:2, :]) ** 2
         + (q[:, 2:3] - xT[2:3, :]) ** 2)
    Qb = q.shape[0]
    iota = lax.broadcasted_iota(jnp.int32, (Qb, V), 1)
    val = jnp.where(d > r2, V, iota)
    t0, t1 = _split_bf16(feat_ref[0])
    out = None
    g0 = None
    for k in range(nsample):
        m = jnp.min(val, axis=1, keepdims=True)
        sel = val == m
        val = jnp.where(sel, V, val)
        g = _gdot(sel, t0, t1)
        if k == 0:
            g0 = g
        else:
            g = jnp.where(m == V, g0, g)
        out = g if out is None else jnp.maximum(out, g)
    o_ref[0] = out


def _pool(queries, xT, feat, radius, nsample, qb):
    B, S, _ = queries.shape
    V, C = feat.shape[1], feat.shape[2]
    body = functools.partial(_pool_body, r2=radius * radius,
                             nsample=nsample, V=V)
    return _pc(
        body,
        grid=(B, S // qb),
        in_specs=[
            pl.BlockSpec((1, qb, 3), lambda b, i: (b, i, 0)),
            pl.BlockSpec((1, 3, V), lambda b, i: (b, 0, 0)),
            pl.BlockSpec((1, V, C), lambda b, i: (b, 0, 0)),
        ],
        out_specs=pl.BlockSpec((1, qb, C), lambda b, i: (b, i, 0)),
        out_shape=jax.ShapeDtypeStruct((B, S, C), jnp.float32),
    )(queries, xT, feat)


def _upsample_body(t_ref, sT_ref, feat_ref, o_ref, *, S):
    t = t_ref[0]
    sT = sT_ref[0]
    d = ((t[:, 0:1] - sT[0:1, :]) ** 2
         + (t[:, 1:2] - sT[1:2, :]) ** 2
         + (t[:, 2:3] - sT[2:3, :]) ** 2)
    Tb = t.shape[0]
    m = jnp.min(d, axis=1, keepdims=True)
    iota = lax.broadcasted_iota(jnp.int32, (Tb, S), 1)
    idx = jnp.min(jnp.where(d == m, iota, S), axis=1, keepdims=True)
    t0, t1 = _split_bf16(feat_ref[0])
    o_ref[0] = _gdot(iota == idx, t0, t1)


def _upsample(targets, sT, feat, qb):
    B, T, _ = targets.shape
    S, C = feat.shape[1], feat.shape[2]
    body = functools.partial(_upsample_body, S=S)
    return _pc(
        body,
        grid=(B, T // qb),
        in_specs=[
            pl.BlockSpec((1, qb, 3), lambda b, i: (b, i, 0)),
            pl.BlockSpec((1, 3, S), lambda b, i: (b, 0, 0)),
            pl.BlockSpec((1, S, C), lambda b, i: (b, 0, 0)),
        ],
        out_specs=pl.BlockSpec((1, qb, C), lambda b, i: (b, i, 0)),
        out_shape=jax.ShapeDtypeStruct((B, T, C), jnp.float32),
    )(targets, sT, feat)


def _mlp_body(x_ref, w1_ref, b1_ref, w2_ref, b2_ref, w3_ref, b3_ref, o_ref):
    h = jnp.maximum(_hdot(x_ref[0], w1_ref[...]) + b1_ref[...], 0.0)
    h = jnp.maximum(_hdot(h, w2_ref[...]) + b2_ref[...], 0.0)
    o_ref[0] = _hdot(h, w3_ref[...]) + b3_ref[...]


def _mlp(x, W1, B1, W2, B2, W3, B3, qb):
    B, T, C = x.shape
    H1, H2, CO = W1.shape[1], W2.shape[1], W3.shape[1]
    return _pc(
        _mlp_body,
        grid=(B, T // qb),
        in_specs=[
            pl.BlockSpec((1, qb, C), lambda b, i: (b, i, 0)),
            pl.BlockSpec((C, H1), lambda b, i: (0, 0)),
            pl.BlockSpec((1, H1), lambda b, i: (0, 0)),
            pl.BlockSpec((H1, H2), lambda b, i: (0, 0)),
            pl.BlockSpec((1, H2), lambda b, i: (0, 0)),
            pl.BlockSpec((H2, CO), lambda b, i: (0, 0)),
            pl.BlockSpec((1, CO), lambda b, i: (0, 0)),
        ],
        out_specs=pl.BlockSpec((1, qb, CO), lambda b, i: (b, i, 0)),
        out_shape=jax.ShapeDtypeStruct((B, T, CO), jnp.float32),
    )(x, W1, B1.reshape(1, H1), W2, B2.reshape(1, H2), W3, B3.reshape(1, CO))


def kernel(vertices, onehot, dir0, w1, b1, dir1, w2, b2, dir2, w3, b3, dir3,
           w4, b4, dir4, w5, b5, dir5, w6, b6, dir6, w7, b7, dir7,
           W1, B1, W2, B2, W3, B3):
    B, N, _ = vertices.shape
    xyz0 = vertices
    xT0 = jnp.transpose(xyz0, (0, 2, 1))

    ni0, dx0, dy0, dz0, nioff, fm0 = _ball_dn(xyz0, xT0, 0.25, NBR, qb=512,
                                              sup=dir0, with_off=True)
    idx0 = jnp.transpose(nioff, (2, 0, 1)).reshape(NBR * B * N)
    fo1 = _fo_full([fm0], w1, b1)
    g1 = _sc_gather(fo1.reshape(B * N, fo1.shape[2]), idx0)
    fm1 = _conv_agg(dx0, dy0, dz0, g1.reshape(NBR, B, N, -1), fo1, dir1,
                    qb=256)
    fo2 = _fo_full([fm0, fm1], w2, b2)
    g2 = _sc_gather(fo2.reshape(B * N, fo2.shape[2]), idx0)
    fm2 = _conv_agg(dx0, dy0, dz0, g2.reshape(NBR, B, N, -1), fo2, dir2,
                    qb=256)
    fm2c = jnp.concatenate([fm0, fm1, fm2], axis=2)

    vp1 = xyz0[:, ::4, :]
    fp1 = _pool(vp1, xT0, fm2c, 0.25, 4, qb=512)
    xT1 = jnp.transpose(vp1, (0, 2, 1))
    ni1, dx1, dy1, dz1 = _ball_dn(vp1, xT1, 0.39, NBR, qb=512)
    fm3, fm4, fm5 = _stage_convs(ni1, dx1, dy1, dz1, fp1,
                                 [(w3, b3, dir3), (w4, b4, dir4),
                                  (w5, b5, dir5)])
    fm5c = jnp.concatenate([fp1, fm3, fm4, fm5], axis=2)

    vp2 = vp1[:, ::4, :]
    fp2 = _pool(vp2, xT1, fm5c, 0.39, 4, qb=128)
    xT2 = jnp.transpose(vp2, (0, 2, 1))
    ni2, dx2, dy2, dz2 = _ball_dn(vp2, xT2, 0.63, NBR, qb=128)
    fm6, fm7, g_fp2, g6, g7 = _stage_convs(ni2, dx2, dy2, dz2, fp2,
                                           [(w6, b6, dir6), (w7, b7, dir7)],
                                           with_gmax=True)
    fm7c = jnp.concatenate([fp2, fm6, fm7], axis=2)
    fglob = jnp.concatenate([g_fp2, g6, g7], axis=2)

    catA = jnp.concatenate([fp1, fm3, fp1, fm3, fm4, fp1, fm3, fm4, fm5],
                           axis=2)
    catB = jnp.concatenate([fp2, fm6, fm7c], axis=2)
    upA = _upsample(xyz0, xT1, catA, qb=512)
    upB = _upsample(xyz0, xT2, catB, qb=512)

    fuse = jnp.concatenate([
        fm0, fm0, fm1, fm2c, upA, upB,
        jnp.broadcast_to(fglob, (B, N, fglob.shape[2])),
        jnp.broadcast_to(onehot[:, None, :], (B, N, onehot.shape[1])),
    ], axis=2)
    return _mlp(fuse, W1, B1, W2, B2, W3, B3, qb=512)
